# Initial kernel scaffold; baseline (speedup 1.0000x reference)
import functools

import jax
import jax.numpy as jnp
from jax import lax
from jax.experimental import pallas as pl
from jax.experimental.pallas import tpu as pltpu
from jax.experimental.pallas import tpu_sc as plsc

_N = 10000
_E = 320000
_D = 128
_L = 200
_A1 = 600
_NCLS = 16
_NM = 5

_CHUNK = 125
_NCHUNKS = _E // _CHUNK
_TILES = 16
_CPT = _NCHUNKS // _TILES
_RPT = _N // _TILES
_ROWBLK = 500
_FINBLK = 1000


def _qbias_body(bw_ref, fc_ref, bb_ref, fcb_ref, q_ref, bias_ref):
    g = [
        jnp.dot(bw_ref[i], fc_ref[_L * i:_L * (i + 1), :],
                preferred_element_type=jnp.float32)
        for i in range(3)
    ]
    zero = jnp.zeros((_L, _NCLS), jnp.float32)
    cols = []
    for k in range(_NM):
        parts = []
        for b in range(3):
            i = k - b
            parts.append(g[i][_L * b:_L * (b + 1), :] if 0 <= i <= 2 else zero)
        cols.append(jnp.concatenate(parts, axis=0))
    q_ref[...] = jnp.concatenate(cols, axis=1)
    bias = fcb_ref[...]
    for i in range(3):
        bias = bias + jnp.dot(bb_ref[i], fc_ref[_L * i:_L * (i + 1), :],
                              preferred_element_type=jnp.float32)
    bias_ref[...] = bias


def _dense_body(f_ref, w_ref, b_ref, q_ref, *m_refs):
    h = jnp.dot(f_ref[...], w_ref[...], preferred_element_type=jnp.float32)
    h = jnp.maximum(h + b_ref[...], 0.0)
    m = jnp.dot(h, q_ref[...], preferred_element_type=jnp.float32)
    for k in range(_NM):
        m_refs[k][...] = m[:, _NCLS * k:_NCLS * (k + 1)]


def _finish_body(x_ref, b_ref, o_ref):
    z = x_ref[...] + b_ref[...]
    z = z - jnp.max(z, axis=1, keepdims=True)
    o_ref[...] = z - jnp.log(jnp.sum(jnp.exp(z), axis=1, keepdims=True))


def _sc_chain_body(cols_ref, rows_ref, vals_ref, m0_ref, m1_ref, m2_ref,
                   m3_ref, m4_ref, zeros_ref, out_ref, pa_ref, pb_ref,
                   cols_v, rows_v, vals_v, rb0, rb1, zbuf, obuf, mbuf,
                   acc, sem0, sem1):
    cid = lax.axis_index("c")
    sid = lax.axis_index("s")

    @pl.when(cid == 0)
    def _core0():
        cbase = sid * _CPT
        rbase = sid * _RPT
        pltpu.sync_copy(cols_ref.at[pl.ds(cbase, _CPT)], cols_v)
        pltpu.sync_copy(rows_ref.at[pl.ds(cbase, _CPT)], rows_v)
        pltpu.sync_copy(vals_ref.at[pl.ds(cbase, _CPT)], vals_v)
        pltpu.sync_copy(zeros_ref, zbuf)

        def scale(rb, c):
            def inner(k, _):
                e0 = k * 25
                for u in range(25):
                    e = e0 + u
                    rb[e, :] = rb[e, :] * vals_v[c, e]
                return 0

            lax.fori_loop(0, _CHUNK // 25, inner, 0)

        def hop(x_src, m_src, x_dst):
            pltpu.sync_copy(zbuf, acc.at[pl.ds(rbase, _RPT)])
            plsc.subcore_barrier()
            pltpu.async_copy(x_src.at[cols_v.at[0]], rb0, sem0)

            def body(j, _):
                c0 = 2 * j
                pltpu.make_async_copy(x_src.at[cols_v.at[c0]], rb0, sem0).wait()
                pltpu.async_copy(x_src.at[cols_v.at[c0 + 1]], rb1, sem1)
                scale(rb0, c0)
                pltpu.sync_copy(rb0, acc.at[rows_v.at[c0]], add=True)
                pltpu.make_async_copy(
                    x_src.at[cols_v.at[c0 + 1]], rb1, sem1).wait()

                @pl.when(j < _CPT // 2 - 1)
                def _issue():
                    pltpu.async_copy(x_src.at[cols_v.at[c0 + 2]], rb0, sem0)

                scale(rb1, c0 + 1)
                pltpu.sync_copy(rb1, acc.at[rows_v.at[c0 + 1]], add=True)
                return 0

            lax.fori_loop(0, _CPT // 2, body, 0)
            plsc.subcore_barrier()
            pltpu.sync_copy(acc.at[pl.ds(rbase, _RPT)], obuf)
            pltpu.sync_copy(m_src.at[pl.ds(rbase, _RPT)], mbuf)

            def radd(k, _):
                e0 = k * 5
                for u in range(5):
                    e = e0 + u
                    obuf[e, :] = obuf[e, :] + mbuf[e, :]
                return 0

            lax.fori_loop(0, _RPT // 5, radd, 0)
            pltpu.sync_copy(obuf, x_dst.at[pl.ds(rbase, _RPT)])
            plsc.subcore_barrier()

        hop(m4_ref, m3_ref, pa_ref)
        hop(pa_ref, m2_ref, pb_ref)
        hop(pb_ref, m1_ref, pa_ref)
        hop(pa_ref, m0_ref, out_ref)


def _sc_chain(cols, rows, vals, m, zeros):
    mesh = plsc.VectorSubcoreMesh(core_axis_name="c", subcore_axis_name="s")
    f = pl.kernel(
        _sc_chain_body,
        out_type=[jax.ShapeDtypeStruct((_N, _NCLS), jnp.float32)] * 3,
        mesh=mesh,
        scratch_types=[
            pltpu.VMEM((_CPT, _CHUNK), jnp.int32),
            pltpu.VMEM((_CPT, _CHUNK), jnp.int32),
            pltpu.VMEM((_CPT, _CHUNK), jnp.float32),
            pltpu.VMEM((_CHUNK, _NCLS), jnp.float32),
            pltpu.VMEM((_CHUNK, _NCLS), jnp.float32),
            pltpu.VMEM((_RPT, _NCLS), jnp.float32),
            pltpu.VMEM((_RPT, _NCLS), jnp.float32),
            pltpu.VMEM((_RPT, _NCLS), jnp.float32),
            pltpu.VMEM_SHARED((_N, _NCLS), jnp.float32),
            pltpu.SemaphoreType.DMA,
            pltpu.SemaphoreType.DMA,
        ],
    )
    return f(cols, rows, vals, m[0], m[1], m[2], m[3], m[4], zeros)[0]


def kernel(features, adj_values, upper_W, upper_b, bottom_W, bottom_b,
           fc_W, fc_b, adj_indices):
    f32 = jnp.float32
    features = features.astype(f32)
    cols = adj_indices[1].astype(jnp.int32).reshape(_NCHUNKS, _CHUNK)
    rows = adj_indices[0].astype(jnp.int32).reshape(_NCHUNKS, _CHUNK)
    vals = adj_values.astype(f32).reshape(_NCHUNKS, _CHUNK)

    w_all = jnp.transpose(upper_W, (1, 0, 2)).reshape(_D, _A1)
    b_all = upper_b.reshape(1, _A1)

    q, bias = pl.pallas_call(
        _qbias_body,
        out_shape=[
            jax.ShapeDtypeStruct((_A1, _NCLS * _NM), f32),
            jax.ShapeDtypeStruct((1, _NCLS), f32),
        ],
    )(bottom_W.astype(f32), fc_W.astype(f32), bottom_b.astype(f32),
      fc_b.astype(f32).reshape(1, _NCLS))

    nblk = _N // _ROWBLK
    m = pl.pallas_call(
        _dense_body,
        grid=(nblk,),
        in_specs=[
            pl.BlockSpec((_ROWBLK, _D), lambda i: (i, 0)),
            pl.BlockSpec((_D, _A1), lambda i: (0, 0)),
            pl.BlockSpec((1, _A1), lambda i: (0, 0)),
            pl.BlockSpec((_A1, _NCLS * _NM), lambda i: (0, 0)),
        ],
        out_specs=[pl.BlockSpec((_ROWBLK, _NCLS), lambda i: (i, 0))] * _NM,
        out_shape=[jax.ShapeDtypeStruct((_N, _NCLS), f32)] * _NM,
    )(features, w_all, b_all, q)

    zeros = jnp.zeros((_RPT, _NCLS), f32)
    p = _sc_chain(cols, rows, vals, m, zeros)

    out = pl.pallas_call(
        _finish_body,
        grid=(_N // _FINBLK,),
        in_specs=[
            pl.BlockSpec((_FINBLK, _NCLS), lambda i: (i, 0)),
            pl.BlockSpec((1, _NCLS), lambda i: (0, 0)),
        ],
        out_specs=pl.BlockSpec((_FINBLK, _NCLS), lambda i: (i, 0)),
        out_shape=jax.ShapeDtypeStruct((_N, _NCLS), f32),
    )(p, bias)
    return out


# speedup vs baseline: 12.8235x; 12.8235x over previous
import functools

import jax
import jax.numpy as jnp
from jax import lax
from jax.experimental import pallas as pl
from jax.experimental.pallas import tpu as pltpu
from jax.experimental.pallas import tpu_sc as plsc

_N = 10000
_E = 320000
_D = 128
_L = 200
_A1 = 600
_NCLS = 16
_NM = 5

_CHUNK = 80
_NCHUNKS = _E // _CHUNK
_TILES = 16
_CPT = _NCHUNKS // _TILES
_NPAD = 10240
_RPT = _NPAD // _TILES
_ROWBLK = 400
_FINBLK = 1000


def _qbias_body(bw_ref, fc_ref, bb_ref, fcb_ref, q_ref, bias_ref):
    g = [
        jnp.dot(bw_ref[i], fc_ref[_L * i:_L * (i + 1), :],
                preferred_element_type=jnp.float32)
        for i in range(3)
    ]
    zero = jnp.zeros((_L, _NCLS), jnp.float32)
    cols = []
    for k in range(_NM):
        parts = []
        for b in range(3):
            i = k - b
            parts.append(g[i][_L * b:_L * (b + 1), :] if 0 <= i <= 2 else zero)
        cols.append(jnp.concatenate(parts, axis=0))
    q_ref[...] = jnp.concatenate(cols, axis=1)
    bias = fcb_ref[...]
    for i in range(3):
        bias = bias + jnp.dot(bb_ref[i], fc_ref[_L * i:_L * (i + 1), :],
                              preferred_element_type=jnp.float32)
    bias_ref[...] = bias


def _dense_body(f_ref, w_ref, b_ref, q_ref, *m_refs):
    h = jnp.dot(f_ref[...], w_ref[...], preferred_element_type=jnp.float32)
    h = jnp.maximum(h + b_ref[...], 0.0)
    m = jnp.dot(h, q_ref[...], preferred_element_type=jnp.float32)
    for k in range(_NM):
        m_refs[k][...] = m[:, _NCLS * k:_NCLS * (k + 1)]


def _finish_body(x_ref, b_ref, o_ref):
    z = x_ref[...] + b_ref[...]
    z = z - jnp.max(z, axis=1, keepdims=True)
    o_ref[...] = z - jnp.log(jnp.sum(jnp.exp(z), axis=1, keepdims=True))


def _sc_chain_body(cols_ref, rows_ref, vals_ref, m0_ref, m1_ref, m2_ref,
                   m3_ref, m4_ref, zeros_ref, out_ref, pa_ref, pb_ref,
                   cols_v, rows_v, vals_v, rb0, rb1, zbuf, obuf, mbuf,
                   acc, sem0, sem1):
    cid = lax.axis_index("c")
    sid = lax.axis_index("s")

    @pl.when(cid == 0)
    def _core0():
        rbase = sid * _RPT
        pltpu.sync_copy(cols_ref.at[sid], cols_v)
        pltpu.sync_copy(rows_ref.at[sid], rows_v)
        pltpu.sync_copy(vals_ref.at[sid], vals_v)
        pltpu.sync_copy(zeros_ref, zbuf)

        def scale(rb, c):
            for g in range(_CHUNK // 16):
                vv = vals_v[c, pl.ds(g * 16, 16)]
                for u in range(16):
                    e = g * 16 + u
                    rb[e, :] = rb[e, :] * vv[u]

        def hop(x_src, m_src, x_dst):
            pltpu.sync_copy(zbuf, acc.at[pl.ds(rbase, _RPT)])
            plsc.subcore_barrier()
            pltpu.async_copy(x_src.at[cols_v.at[0]], rb0, sem0)

            def body(j, _):
                c0 = 2 * j
                pltpu.make_async_copy(x_src.at[cols_v.at[c0]], rb0, sem0).wait()
                pltpu.async_copy(x_src.at[cols_v.at[c0 + 1]], rb1, sem1)
                scale(rb0, c0)
                pltpu.sync_copy(rb0, acc.at[rows_v.at[c0]], add=True)
                pltpu.make_async_copy(
                    x_src.at[cols_v.at[c0 + 1]], rb1, sem1).wait()

                @pl.when(j < _CPT // 2 - 1)
                def _issue():
                    pltpu.async_copy(x_src.at[cols_v.at[c0 + 2]], rb0, sem0)

                scale(rb1, c0 + 1)
                pltpu.sync_copy(rb1, acc.at[rows_v.at[c0 + 1]], add=True)
                return 0

            lax.fori_loop(0, _CPT // 2, body, 0)
            plsc.subcore_barrier()
            pltpu.sync_copy(acc.at[pl.ds(rbase, _RPT)], obuf)
            pltpu.sync_copy(m_src.at[pl.ds(rbase, _RPT)], mbuf)

            def radd(k, _):
                e0 = k * 5
                for u in range(5):
                    e = e0 + u
                    obuf[e, :] = obuf[e, :] + mbuf[e, :]
                return 0

            lax.fori_loop(0, _RPT // 5, radd, 0)
            pltpu.sync_copy(obuf, x_dst.at[pl.ds(rbase, _RPT)])
            plsc.subcore_barrier()

        hop(m4_ref, m3_ref, pa_ref)
        hop(pa_ref, m2_ref, pb_ref)
        hop(pb_ref, m1_ref, pa_ref)
        hop(pa_ref, m0_ref, out_ref)


def _sc_chain(cols, rows, vals, m, zeros):
    mesh = plsc.VectorSubcoreMesh(core_axis_name="c", subcore_axis_name="s")
    f = pl.kernel(
        _sc_chain_body,
        out_type=[jax.ShapeDtypeStruct((_NPAD, _NCLS), jnp.float32)] * 3,
        mesh=mesh,
        compiler_params=pltpu.CompilerParams(use_tc_tiling_on_sc=False),
        scratch_types=[
            pltpu.VMEM((_CPT, _CHUNK), jnp.int32),
            pltpu.VMEM((_CPT, _CHUNK), jnp.int32),
            pltpu.VMEM((_CPT, _CHUNK), jnp.float32),
            pltpu.VMEM((_CHUNK, _NCLS), jnp.float32),
            pltpu.VMEM((_CHUNK, _NCLS), jnp.float32),
            pltpu.VMEM((_RPT, _NCLS), jnp.float32),
            pltpu.VMEM((_RPT, _NCLS), jnp.float32),
            pltpu.VMEM((_RPT, _NCLS), jnp.float32),
            pltpu.VMEM_SHARED((_NPAD, _NCLS), jnp.float32),
            pltpu.SemaphoreType.DMA,
            pltpu.SemaphoreType.DMA,
        ],
    )
    return f(cols, rows, vals, m[0], m[1], m[2], m[3], m[4], zeros)[0]


def kernel(features, adj_values, upper_W, upper_b, bottom_W, bottom_b,
           fc_W, fc_b, adj_indices):
    f32 = jnp.float32
    features = features.astype(f32)
    cols = adj_indices[1].astype(jnp.int32).reshape(_TILES, _CPT, _CHUNK)
    rows = adj_indices[0].astype(jnp.int32).reshape(_TILES, _CPT, _CHUNK)
    vals = adj_values.astype(f32).reshape(_TILES, _CPT, _CHUNK)

    w_all = jnp.transpose(upper_W, (1, 0, 2)).reshape(_D, _A1)
    b_all = upper_b.reshape(1, _A1)

    q, bias = pl.pallas_call(
        _qbias_body,
        out_shape=[
            jax.ShapeDtypeStruct((_A1, _NCLS * _NM), f32),
            jax.ShapeDtypeStruct((1, _NCLS), f32),
        ],
    )(bottom_W.astype(f32), fc_W.astype(f32), bottom_b.astype(f32),
      fc_b.astype(f32).reshape(1, _NCLS))

    nblk = _N // _ROWBLK
    m = pl.pallas_call(
        _dense_body,
        grid=(nblk,),
        in_specs=[
            pl.BlockSpec((_ROWBLK, _D), lambda i: (i, 0)),
            pl.BlockSpec((_D, _A1), lambda i: (0, 0)),
            pl.BlockSpec((1, _A1), lambda i: (0, 0)),
            pl.BlockSpec((_A1, _NCLS * _NM), lambda i: (0, 0)),
        ],
        out_specs=[pl.BlockSpec((_ROWBLK, _NCLS), lambda i: (i, 0))] * _NM,
        out_shape=[jax.ShapeDtypeStruct((_NPAD, _NCLS), f32)] * _NM,
    )(features, w_all, b_all, q)

    zeros = jnp.zeros((_RPT, _NCLS), f32)
    p = _sc_chain(cols, rows, vals, m, zeros)

    out = pl.pallas_call(
        _finish_body,
        grid=(_N // _FINBLK,),
        in_specs=[
            pl.BlockSpec((_FINBLK, _NCLS), lambda i: (i, 0)),
            pl.BlockSpec((1, _NCLS), lambda i: (0, 0)),
        ],
        out_specs=pl.BlockSpec((_FINBLK, _NCLS), lambda i: (i, 0)),
        out_shape=jax.ShapeDtypeStruct((_N, _NCLS), f32),
    )(p, bias)
    return out


# speedup vs baseline: 13.8068x; 1.0767x over previous
import functools

import jax
import jax.numpy as jnp
from jax import lax
from jax.experimental import pallas as pl
from jax.experimental.pallas import tpu as pltpu
from jax.experimental.pallas import tpu_sc as plsc

_N = 10000
_E = 320000
_D = 128
_L = 200
_A1 = 600
_NCLS = 16
_NM = 5

_CHUNK = 125
_NCHUNKS = _E // _CHUNK
_TILES = 16
_CPT = _NCHUNKS // _TILES
_NPAD = 10240
_RPT = _NPAD // _TILES
_ROWBLK = 400
_FINBLK = 1000


def _qbias_body(bw_ref, fc_ref, bb_ref, fcb_ref, q_ref, bias_ref):
    g = [
        jnp.dot(bw_ref[i], fc_ref[_L * i:_L * (i + 1), :],
                preferred_element_type=jnp.float32)
        for i in range(3)
    ]
    zero = jnp.zeros((_L, _NCLS), jnp.float32)
    cols = []
    for k in range(_NM):
        parts = []
        for b in range(3):
            i = k - b
            parts.append(g[i][_L * b:_L * (b + 1), :] if 0 <= i <= 2 else zero)
        cols.append(jnp.concatenate(parts, axis=0))
    q_ref[...] = jnp.concatenate(cols, axis=1)
    bias = fcb_ref[...]
    for i in range(3):
        bias = bias + jnp.dot(bb_ref[i], fc_ref[_L * i:_L * (i + 1), :],
                              preferred_element_type=jnp.float32)
    bias_ref[...] = bias


def _dense_body(f_ref, w_ref, b_ref, q_ref, *m_refs):
    h = jnp.dot(f_ref[...], w_ref[...], preferred_element_type=jnp.float32)
    h = jnp.maximum(h + b_ref[...], 0.0)
    m = jnp.dot(h, q_ref[...], preferred_element_type=jnp.float32)
    for k in range(_NM):
        m_refs[k][...] = m[:, _NCLS * k:_NCLS * (k + 1)]


def _finish_body(x_ref, b_ref, o_ref):
    z = x_ref[...] + b_ref[...]
    z = z - jnp.max(z, axis=1, keepdims=True)
    o_ref[...] = z - jnp.log(jnp.sum(jnp.exp(z), axis=1, keepdims=True))


def _sc_chain_body(cols_ref, rows_ref, vals_ref, m0_ref, m1_ref, m2_ref,
                   m3_ref, m4_ref, zeros_ref, out_ref, pa_ref, pb_ref,
                   cols_v, rows_v, rb0, rb1, vb0, vb1, zbuf, obuf, mbuf,
                   acc, sem0, sem1, sem2, sem3):
    cid = lax.axis_index("c")
    sid = lax.axis_index("s")

    @pl.when(cid == 0)
    def _core0():
        rbase = sid * _RPT
        pltpu.sync_copy(cols_ref.at[sid], cols_v)
        pltpu.sync_copy(rows_ref.at[sid], rows_v)
        pltpu.sync_copy(zeros_ref, zbuf)

        def scale(rb, vb):
            for e in range(_CHUNK):
                rb[e, :] = rb[e, :] * vb[e, :]

        def hop(x_src, m_src, x_dst):
            pltpu.sync_copy(zbuf, acc.at[pl.ds(rbase, _RPT)])
            plsc.subcore_barrier()
            pltpu.async_copy(x_src.at[cols_v.at[0]], rb0, sem0)
            pltpu.async_copy(vals_ref.at[sid, 0], vb0, sem2)

            def body(j, _):
                c0 = 2 * j
                pltpu.make_async_copy(x_src.at[cols_v.at[c0]], rb0, sem0).wait()
                pltpu.make_async_copy(vals_ref.at[sid, c0], vb0, sem2).wait()
                pltpu.async_copy(x_src.at[cols_v.at[c0 + 1]], rb1, sem1)
                pltpu.async_copy(vals_ref.at[sid, c0 + 1], vb1, sem3)
                scale(rb0, vb0)
                pltpu.sync_copy(rb0, acc.at[rows_v.at[c0]], add=True)
                pltpu.make_async_copy(
                    x_src.at[cols_v.at[c0 + 1]], rb1, sem1).wait()
                pltpu.make_async_copy(
                    vals_ref.at[sid, c0 + 1], vb1, sem3).wait()

                @pl.when(j < _CPT // 2 - 1)
                def _issue():
                    pltpu.async_copy(x_src.at[cols_v.at[c0 + 2]], rb0, sem0)
                    pltpu.async_copy(vals_ref.at[sid, c0 + 2], vb0, sem2)

                scale(rb1, vb1)
                pltpu.sync_copy(rb1, acc.at[rows_v.at[c0 + 1]], add=True)
                return 0

            lax.fori_loop(0, _CPT // 2, body, 0)
            plsc.subcore_barrier()
            pltpu.sync_copy(acc.at[pl.ds(rbase, _RPT)], obuf)
            pltpu.sync_copy(m_src.at[pl.ds(rbase, _RPT)], mbuf)

            def radd(k, _):
                e0 = k * 5
                for u in range(5):
                    e = e0 + u
                    obuf[e, :] = obuf[e, :] + mbuf[e, :]
                return 0

            lax.fori_loop(0, _RPT // 5, radd, 0)
            pltpu.sync_copy(obuf, x_dst.at[pl.ds(rbase, _RPT)])
            plsc.subcore_barrier()

        hop(m4_ref, m3_ref, pa_ref)
        hop(pa_ref, m2_ref, pb_ref)
        hop(pb_ref, m1_ref, pa_ref)
        hop(pa_ref, m0_ref, out_ref)


def _sc_chain(cols, rows, vals, m, zeros):
    mesh = plsc.VectorSubcoreMesh(core_axis_name="c", subcore_axis_name="s")
    f = pl.kernel(
        _sc_chain_body,
        out_type=[jax.ShapeDtypeStruct((_NPAD, _NCLS), jnp.float32)] * 3,
        mesh=mesh,
        compiler_params=pltpu.CompilerParams(use_tc_tiling_on_sc=False),
        scratch_types=[
            pltpu.VMEM((_CPT, _CHUNK), jnp.int32),
            pltpu.VMEM((_CPT, _CHUNK), jnp.int32),
            pltpu.VMEM((_CHUNK, _NCLS), jnp.float32),
            pltpu.VMEM((_CHUNK, _NCLS), jnp.float32),
            pltpu.VMEM((_CHUNK, _NCLS), jnp.float32),
            pltpu.VMEM((_CHUNK, _NCLS), jnp.float32),
            pltpu.VMEM((_RPT, _NCLS), jnp.float32),
            pltpu.VMEM((_RPT, _NCLS), jnp.float32),
            pltpu.VMEM((_RPT, _NCLS), jnp.float32),
            pltpu.VMEM_SHARED((_NPAD, _NCLS), jnp.float32),
            pltpu.SemaphoreType.DMA,
            pltpu.SemaphoreType.DMA,
            pltpu.SemaphoreType.DMA,
            pltpu.SemaphoreType.DMA,
        ],
    )
    return f(cols, rows, vals, m[0], m[1], m[2], m[3], m[4], zeros)[0]


def kernel(features, adj_values, upper_W, upper_b, bottom_W, bottom_b,
           fc_W, fc_b, adj_indices):
    f32 = jnp.float32
    features = features.astype(f32)
    cols = adj_indices[1].astype(jnp.int32).reshape(_TILES, _CPT, _CHUNK)
    rows = adj_indices[0].astype(jnp.int32).reshape(_TILES, _CPT, _CHUNK)
    vals = jnp.broadcast_to(
        adj_values.astype(f32).reshape(_TILES, _CPT, _CHUNK)[..., None],
        (_TILES, _CPT, _CHUNK, _NCLS))

    w_all = jnp.transpose(upper_W, (1, 0, 2)).reshape(_D, _A1)
    b_all = upper_b.reshape(1, _A1)

    q, bias = pl.pallas_call(
        _qbias_body,
        out_shape=[
            jax.ShapeDtypeStruct((_A1, _NCLS * _NM), f32),
            jax.ShapeDtypeStruct((1, _NCLS), f32),
        ],
    )(bottom_W.astype(f32), fc_W.astype(f32), bottom_b.astype(f32),
      fc_b.astype(f32).reshape(1, _NCLS))

    nblk = _N // _ROWBLK
    m = pl.pallas_call(
        _dense_body,
        grid=(nblk,),
        in_specs=[
            pl.BlockSpec((_ROWBLK, _D), lambda i: (i, 0)),
            pl.BlockSpec((_D, _A1), lambda i: (0, 0)),
            pl.BlockSpec((1, _A1), lambda i: (0, 0)),
            pl.BlockSpec((_A1, _NCLS * _NM), lambda i: (0, 0)),
        ],
        out_specs=[pl.BlockSpec((_ROWBLK, _NCLS), lambda i: (i, 0))] * _NM,
        out_shape=[jax.ShapeDtypeStruct((_NPAD, _NCLS), f32)] * _NM,
    )(features, w_all, b_all, q)

    zeros = jnp.zeros((_RPT, _NCLS), f32)
    p = _sc_chain(cols, rows, vals, m, zeros)

    out = pl.pallas_call(
        _finish_body,
        grid=(_N // _FINBLK,),
        in_specs=[
            pl.BlockSpec((_FINBLK, _NCLS), lambda i: (i, 0)),
            pl.BlockSpec((1, _NCLS), lambda i: (0, 0)),
        ],
        out_specs=pl.BlockSpec((_FINBLK, _NCLS), lambda i: (i, 0)),
        out_shape=jax.ShapeDtypeStruct((_N, _NCLS), f32),
    )(p, bias)
    return out


# speedup vs baseline: 13.8205x; 1.0010x over previous
import functools

import jax
import jax.numpy as jnp
from jax import lax
from jax.experimental import pallas as pl
from jax.experimental.pallas import tpu as pltpu
from jax.experimental.pallas import tpu_sc as plsc

_N = 10000
_E = 320000
_D = 128
_L = 200
_A1 = 600
_NCLS = 16
_NM = 5

_CHUNK = 125
_NCHUNKS = _E // _CHUNK
_TILES = 16
_CPT = _NCHUNKS // _TILES
_NPAD = 10240
_RPT = _NPAD // _TILES
_ROWBLK = 400
_FINBLK = 1000


def _qbias_body(bw_ref, fc_ref, bb_ref, fcb_ref, q_ref, bias_ref):
    g = [
        jnp.dot(bw_ref[i], fc_ref[_L * i:_L * (i + 1), :],
                preferred_element_type=jnp.float32)
        for i in range(3)
    ]
    zero = jnp.zeros((_L, _NCLS), jnp.float32)
    cols = []
    for k in range(_NM):
        parts = []
        for b in range(3):
            i = k - b
            parts.append(g[i][_L * b:_L * (b + 1), :] if 0 <= i <= 2 else zero)
        cols.append(jnp.concatenate(parts, axis=0))
    q_ref[...] = jnp.concatenate(cols, axis=1)
    bias = fcb_ref[...]
    for i in range(3):
        bias = bias + jnp.dot(bb_ref[i], fc_ref[_L * i:_L * (i + 1), :],
                              preferred_element_type=jnp.float32)
    bias_ref[...] = bias


def _dense_body(f_ref, w_ref, b_ref, q_ref, *m_refs):
    h = jnp.dot(f_ref[...], w_ref[...], preferred_element_type=jnp.float32)
    h = jnp.maximum(h + b_ref[...], 0.0)
    m = jnp.dot(h, q_ref[...], preferred_element_type=jnp.float32)
    for k in range(_NM):
        m_refs[k][...] = m[:, _NCLS * k:_NCLS * (k + 1)]


def _finish_body(x_ref, b_ref, o_ref):
    z = x_ref[...] + b_ref[...]
    z = z - jnp.max(z, axis=1, keepdims=True)
    o_ref[...] = z - jnp.log(jnp.sum(jnp.exp(z), axis=1, keepdims=True))


def _sc_chain_body(cols_ref, rows_ref, vals_ref, m0_ref, m1_ref, m2_ref,
                   m3_ref, m4_ref, zeros_ref, out_ref, pa_ref, pb_ref,
                   cols_v, rows_v, rb0, rb1, vb0, vb1, zbuf, obuf, mbuf,
                   acc, sem0, sem1, sem2, sem3):
    cid = lax.axis_index("c")
    sid = lax.axis_index("s")

    @pl.when(cid == 0)
    def _core0():
        rbase = sid * _RPT
        pltpu.sync_copy(cols_ref.at[sid], cols_v)
        pltpu.sync_copy(rows_ref.at[sid], rows_v)
        pltpu.sync_copy(zeros_ref, zbuf)

        def scale(rb, vb):
            for e in range(_CHUNK):
                rb[e, :] = rb[e, :] * vb[e, :]

        def hop(x_src, m_src, x_dst, tag):
          with jax.named_scope(tag):
            pltpu.sync_copy(zbuf, acc.at[pl.ds(rbase, _RPT)])
            plsc.subcore_barrier()
            pltpu.async_copy(x_src.at[cols_v.at[0]], rb0, sem0)
            pltpu.async_copy(vals_ref.at[sid, 0], vb0, sem2)

            def body(j, _):
                c0 = 2 * j
                pltpu.make_async_copy(x_src.at[cols_v.at[c0]], rb0, sem0).wait()
                pltpu.make_async_copy(vals_ref.at[sid, c0], vb0, sem2).wait()
                pltpu.async_copy(x_src.at[cols_v.at[c0 + 1]], rb1, sem1)
                pltpu.async_copy(vals_ref.at[sid, c0 + 1], vb1, sem3)
                scale(rb0, vb0)
                pltpu.sync_copy(rb0, acc.at[rows_v.at[c0]], add=True)
                pltpu.make_async_copy(
                    x_src.at[cols_v.at[c0 + 1]], rb1, sem1).wait()
                pltpu.make_async_copy(
                    vals_ref.at[sid, c0 + 1], vb1, sem3).wait()

                @pl.when(j < _CPT // 2 - 1)
                def _issue():
                    pltpu.async_copy(x_src.at[cols_v.at[c0 + 2]], rb0, sem0)
                    pltpu.async_copy(vals_ref.at[sid, c0 + 2], vb0, sem2)

                scale(rb1, vb1)
                pltpu.sync_copy(rb1, acc.at[rows_v.at[c0 + 1]], add=True)
                return 0

            lax.fori_loop(0, _CPT // 2, body, 0)
            plsc.subcore_barrier()
          with jax.named_scope(tag + "_readout"):
            pltpu.sync_copy(acc.at[pl.ds(rbase, _RPT)], obuf)
            pltpu.sync_copy(m_src.at[pl.ds(rbase, _RPT)], mbuf)

            def radd(k, _):
                e0 = k * 5
                for u in range(5):
                    e = e0 + u
                    obuf[e, :] = obuf[e, :] + mbuf[e, :]
                return 0

            lax.fori_loop(0, _RPT // 5, radd, 0)
            pltpu.sync_copy(obuf, x_dst.at[pl.ds(rbase, _RPT)])
            plsc.subcore_barrier()

        hop(m4_ref, m3_ref, pa_ref, "hop1")
        hop(pa_ref, m2_ref, pb_ref, "hop2")
        hop(pb_ref, m1_ref, pa_ref, "hop3")
        hop(pa_ref, m0_ref, out_ref, "hop4")


def _sc_chain(cols, rows, vals, m, zeros):
    mesh = plsc.VectorSubcoreMesh(core_axis_name="c", subcore_axis_name="s")
    f = pl.kernel(
        _sc_chain_body,
        out_type=[jax.ShapeDtypeStruct((_NPAD, _NCLS), jnp.float32)] * 3,
        mesh=mesh,
        compiler_params=pltpu.CompilerParams(use_tc_tiling_on_sc=False),
        scratch_types=[
            pltpu.VMEM((_CPT, _CHUNK), jnp.int32),
            pltpu.VMEM((_CPT, _CHUNK), jnp.int32),
            pltpu.VMEM((_CHUNK, _NCLS), jnp.float32),
            pltpu.VMEM((_CHUNK, _NCLS), jnp.float32),
            pltpu.VMEM((_CHUNK, _NCLS), jnp.float32),
            pltpu.VMEM((_CHUNK, _NCLS), jnp.float32),
            pltpu.VMEM((_RPT, _NCLS), jnp.float32),
            pltpu.VMEM((_RPT, _NCLS), jnp.float32),
            pltpu.VMEM((_RPT, _NCLS), jnp.float32),
            pltpu.VMEM_SHARED((_NPAD, _NCLS), jnp.float32),
            pltpu.SemaphoreType.DMA,
            pltpu.SemaphoreType.DMA,
            pltpu.SemaphoreType.DMA,
            pltpu.SemaphoreType.DMA,
        ],
    )
    return f(cols, rows, vals, m[0], m[1], m[2], m[3], m[4], zeros)[0]


def kernel(features, adj_values, upper_W, upper_b, bottom_W, bottom_b,
           fc_W, fc_b, adj_indices):
    f32 = jnp.float32
    features = features.astype(f32)
    cols = adj_indices[1].astype(jnp.int32).reshape(_TILES, _CPT, _CHUNK)
    rows = adj_indices[0].astype(jnp.int32).reshape(_TILES, _CPT, _CHUNK)
    vals = jnp.broadcast_to(
        adj_values.astype(f32).reshape(_TILES, _CPT, _CHUNK)[..., None],
        (_TILES, _CPT, _CHUNK, _NCLS))

    w_all = jnp.transpose(upper_W, (1, 0, 2)).reshape(_D, _A1)
    b_all = upper_b.reshape(1, _A1)

    q, bias = pl.pallas_call(
        _qbias_body,
        out_shape=[
            jax.ShapeDtypeStruct((_A1, _NCLS * _NM), f32),
            jax.ShapeDtypeStruct((1, _NCLS), f32),
        ],
    )(bottom_W.astype(f32), fc_W.astype(f32), bottom_b.astype(f32),
      fc_b.astype(f32).reshape(1, _NCLS))

    nblk = _N // _ROWBLK
    m = pl.pallas_call(
        _dense_body,
        grid=(nblk,),
        in_specs=[
            pl.BlockSpec((_ROWBLK, _D), lambda i: (i, 0)),
            pl.BlockSpec((_D, _A1), lambda i: (0, 0)),
            pl.BlockSpec((1, _A1), lambda i: (0, 0)),
            pl.BlockSpec((_A1, _NCLS * _NM), lambda i: (0, 0)),
        ],
        out_specs=[pl.BlockSpec((_ROWBLK, _NCLS), lambda i: (i, 0))] * _NM,
        out_shape=[jax.ShapeDtypeStruct((_NPAD, _NCLS), f32)] * _NM,
    )(features, w_all, b_all, q)

    zeros = jnp.zeros((_RPT, _NCLS), f32)
    p = _sc_chain(cols, rows, vals, m, zeros)

    out = pl.pallas_call(
        _finish_body,
        grid=(_N // _FINBLK,),
        in_specs=[
            pl.BlockSpec((_FINBLK, _NCLS), lambda i: (i, 0)),
            pl.BlockSpec((1, _NCLS), lambda i: (0, 0)),
        ],
        out_specs=pl.BlockSpec((_FINBLK, _NCLS), lambda i: (i, 0)),
        out_shape=jax.ShapeDtypeStruct((_N, _NCLS), f32),
    )(p, bias)
    return out


# speedup vs baseline: 16.7521x; 1.2121x over previous
import functools

import jax
import jax.numpy as jnp
from jax import lax
from jax.experimental import pallas as pl
from jax.experimental.pallas import tpu as pltpu
from jax.experimental.pallas import tpu_sc as plsc

_N = 10000
_E = 320000
_D = 128
_L = 200
_A1 = 600
_NCLS = 16
_NM = 5

_CHUNK = 125
_NCHUNKS = _E // _CHUNK
_TILES = 16
_CPT = _NCHUNKS // _TILES
_NPAD = 10240
_RPT = _NPAD // _TILES
_ROWBLK = 400
_FINBLK = 1000


def _qbias_body(bw_ref, fc_ref, bb_ref, fcb_ref, q_ref, bias_ref):
    g = [
        jnp.dot(bw_ref[i], fc_ref[_L * i:_L * (i + 1), :],
                preferred_element_type=jnp.float32)
        for i in range(3)
    ]
    zero = jnp.zeros((_L, _NCLS), jnp.float32)
    cols = []
    for k in range(_NM):
        parts = []
        for b in range(3):
            i = k - b
            parts.append(g[i][_L * b:_L * (b + 1), :] if 0 <= i <= 2 else zero)
        cols.append(jnp.concatenate(parts, axis=0))
    q_ref[...] = jnp.concatenate(cols, axis=1)
    bias = fcb_ref[...]
    for i in range(3):
        bias = bias + jnp.dot(bb_ref[i], fc_ref[_L * i:_L * (i + 1), :],
                              preferred_element_type=jnp.float32)
    bias_ref[...] = bias


def _dense_body(f_ref, w_ref, b_ref, q_ref, *m_refs):
    h = jnp.dot(f_ref[...], w_ref[...], preferred_element_type=jnp.float32)
    h = jnp.maximum(h + b_ref[...], 0.0)
    m = jnp.dot(h, q_ref[...], preferred_element_type=jnp.float32)
    for k in range(_NM):
        m_refs[k][...] = m[:, _NCLS * k:_NCLS * (k + 1)]


def _finish_body(x_ref, b_ref, o_ref):
    z = x_ref[...] + b_ref[...]
    z = z - jnp.max(z, axis=1, keepdims=True)
    o_ref[...] = z - jnp.log(jnp.sum(jnp.exp(z), axis=1, keepdims=True))


def _sc_chain_body(cols_ref, rows_ref, vals_ref, m0_ref, m1_ref, m2_ref,
                   m3_ref, m4_ref, zeros_ref, out_ref, pa_ref, pb_ref,
                   cols_v, rows_v, vals_v, rb0, rb1, zbuf, obuf, mbuf,
                   acc, sem0, sem1):
    cid = lax.axis_index("c")
    sid = lax.axis_index("s")

    @pl.when(cid == 0)
    def _core0():
        rbase = sid * _RPT
        pltpu.sync_copy(cols_ref.at[sid], cols_v)
        pltpu.sync_copy(rows_ref.at[sid], rows_v)
        pltpu.sync_copy(vals_ref.at[sid], vals_v)
        pltpu.sync_copy(zeros_ref, zbuf)

        lane_ids = [jnp.full((16,), u, jnp.int32) for u in range(16)]

        def scale(rb, c):
            for g in range(8):
                vv = vals_v[c, pl.ds(g * 16, 16)]
                for u in range(16):
                    e = g * 16 + u
                    if e < _CHUNK:
                        rb[e, :] = rb[e, :] * jnp.take(vv, lane_ids[u])

        def hop(x_src, m_src, x_dst, tag):
          with jax.named_scope(tag):
            pltpu.sync_copy(zbuf, acc.at[pl.ds(rbase, _RPT)])
            plsc.subcore_barrier()
            pltpu.async_copy(x_src.at[cols_v.at[0]], rb0, sem0)

            def body(j, _):
                c0 = 2 * j
                pltpu.make_async_copy(x_src.at[cols_v.at[c0]], rb0, sem0).wait()
                pltpu.async_copy(x_src.at[cols_v.at[c0 + 1]], rb1, sem1)
                scale(rb0, c0)
                pltpu.sync_copy(rb0, acc.at[rows_v.at[c0]], add=True)
                pltpu.make_async_copy(
                    x_src.at[cols_v.at[c0 + 1]], rb1, sem1).wait()

                @pl.when(j < _CPT // 2 - 1)
                def _issue():
                    pltpu.async_copy(x_src.at[cols_v.at[c0 + 2]], rb0, sem0)

                scale(rb1, c0 + 1)
                pltpu.sync_copy(rb1, acc.at[rows_v.at[c0 + 1]], add=True)
                return 0

            lax.fori_loop(0, _CPT // 2, body, 0)
            plsc.subcore_barrier()
          with jax.named_scope(tag + "_readout"):
            pltpu.sync_copy(acc.at[pl.ds(rbase, _RPT)], obuf)
            pltpu.sync_copy(m_src.at[pl.ds(rbase, _RPT)], mbuf)

            def radd(k, _):
                e0 = k * 5
                for u in range(5):
                    e = e0 + u
                    obuf[e, :] = obuf[e, :] + mbuf[e, :]
                return 0

            lax.fori_loop(0, _RPT // 5, radd, 0)
            pltpu.sync_copy(obuf, x_dst.at[pl.ds(rbase, _RPT)])
            plsc.subcore_barrier()

        hop(m4_ref, m3_ref, pa_ref, "hop1")
        hop(pa_ref, m2_ref, pb_ref, "hop2")
        hop(pb_ref, m1_ref, pa_ref, "hop3")
        hop(pa_ref, m0_ref, out_ref, "hop4")


def _sc_chain(cols, rows, vals, m, zeros):
    mesh = plsc.VectorSubcoreMesh(core_axis_name="c", subcore_axis_name="s")
    f = pl.kernel(
        _sc_chain_body,
        out_type=[jax.ShapeDtypeStruct((_NPAD, _NCLS), jnp.float32)] * 3,
        mesh=mesh,
        compiler_params=pltpu.CompilerParams(use_tc_tiling_on_sc=False),
        scratch_types=[
            pltpu.VMEM((_CPT, _CHUNK), jnp.int32),
            pltpu.VMEM((_CPT, _CHUNK), jnp.int32),
            pltpu.VMEM((_CPT, 128), jnp.float32),
            pltpu.VMEM((_CHUNK, _NCLS), jnp.float32),
            pltpu.VMEM((_CHUNK, _NCLS), jnp.float32),
            pltpu.VMEM((_RPT, _NCLS), jnp.float32),
            pltpu.VMEM((_RPT, _NCLS), jnp.float32),
            pltpu.VMEM((_RPT, _NCLS), jnp.float32),
            pltpu.VMEM_SHARED((_NPAD, _NCLS), jnp.float32),
            pltpu.SemaphoreType.DMA,
            pltpu.SemaphoreType.DMA,
        ],
    )
    return f(cols, rows, vals, m[0], m[1], m[2], m[3], m[4], zeros)[0]


def kernel(features, adj_values, upper_W, upper_b, bottom_W, bottom_b,
           fc_W, fc_b, adj_indices):
    f32 = jnp.float32
    features = features.astype(f32)
    cols = adj_indices[1].astype(jnp.int32).reshape(_TILES, _CPT, _CHUNK)
    rows = adj_indices[0].astype(jnp.int32).reshape(_TILES, _CPT, _CHUNK)
    vals = jnp.pad(adj_values.astype(f32).reshape(_TILES, _CPT, _CHUNK),
                   ((0, 0), (0, 0), (0, 128 - _CHUNK)))

    w_all = jnp.transpose(upper_W, (1, 0, 2)).reshape(_D, _A1)
    b_all = upper_b.reshape(1, _A1)

    q, bias = pl.pallas_call(
        _qbias_body,
        out_shape=[
            jax.ShapeDtypeStruct((_A1, _NCLS * _NM), f32),
            jax.ShapeDtypeStruct((1, _NCLS), f32),
        ],
    )(bottom_W.astype(f32), fc_W.astype(f32), bottom_b.astype(f32),
      fc_b.astype(f32).reshape(1, _NCLS))

    nblk = _N // _ROWBLK
    m = pl.pallas_call(
        _dense_body,
        grid=(nblk,),
        in_specs=[
            pl.BlockSpec((_ROWBLK, _D), lambda i: (i, 0)),
            pl.BlockSpec((_D, _A1), lambda i: (0, 0)),
            pl.BlockSpec((1, _A1), lambda i: (0, 0)),
            pl.BlockSpec((_A1, _NCLS * _NM), lambda i: (0, 0)),
        ],
        out_specs=[pl.BlockSpec((_ROWBLK, _NCLS), lambda i: (i, 0))] * _NM,
        out_shape=[jax.ShapeDtypeStruct((_NPAD, _NCLS), f32)] * _NM,
    )(features, w_all, b_all, q)

    zeros = jnp.zeros((_RPT, _NCLS), f32)
    p = _sc_chain(cols, rows, vals, m, zeros)

    out = pl.pallas_call(
        _finish_body,
        grid=(_N // _FINBLK,),
        in_specs=[
            pl.BlockSpec((_FINBLK, _NCLS), lambda i: (i, 0)),
            pl.BlockSpec((1, _NCLS), lambda i: (0, 0)),
        ],
        out_specs=pl.BlockSpec((_FINBLK, _NCLS), lambda i: (i, 0)),
        out_shape=jax.ShapeDtypeStruct((_N, _NCLS), f32),
    )(p, bias)
    return out


# speedup vs baseline: 24.4607x; 1.4602x over previous
import functools

import jax
import jax.numpy as jnp
from jax import lax
from jax.experimental import pallas as pl
from jax.experimental.pallas import tpu as pltpu
from jax.experimental.pallas import tpu_sc as plsc

_N = 10000
_E = 320000
_D = 128
_L = 200
_A1 = 600
_NCLS = 16
_NM = 5

_CHUNK = 125
_NCHUNKS = _E // _CHUNK
_TILES = 16
_CPT = _NCHUNKS // _TILES
_NPAD = 10240
_RPT = _NPAD // _TILES
_ROWBLK = 400
_FINBLK = 1000


def _qbias_body(bw_ref, fc_ref, bb_ref, fcb_ref, q_ref, bias_ref):
    g = [
        jnp.dot(bw_ref[i], fc_ref[_L * i:_L * (i + 1), :],
                preferred_element_type=jnp.float32)
        for i in range(3)
    ]
    zero = jnp.zeros((_L, _NCLS), jnp.float32)
    cols = []
    for k in range(_NM):
        parts = []
        for b in range(3):
            i = k - b
            parts.append(g[i][_L * b:_L * (b + 1), :] if 0 <= i <= 2 else zero)
        cols.append(jnp.concatenate(parts, axis=0))
    q_ref[...] = jnp.concatenate(cols, axis=1)
    bias = fcb_ref[...]
    for i in range(3):
        bias = bias + jnp.dot(bb_ref[i], fc_ref[_L * i:_L * (i + 1), :],
                              preferred_element_type=jnp.float32)
    bias_ref[...] = bias


def _dense_body(f_ref, w_ref, b_ref, q_ref, *m_refs):
    h = jnp.dot(f_ref[...], w_ref[...], preferred_element_type=jnp.float32)
    h = jnp.maximum(h + b_ref[...], 0.0)
    m = jnp.dot(h, q_ref[...], preferred_element_type=jnp.float32)
    for k in range(_NM):
        m_refs[k][...] = m[:, _NCLS * k:_NCLS * (k + 1)]


def _finish_body(x_ref, b_ref, o_ref):
    z = x_ref[...] + b_ref[...]
    z = z - jnp.max(z, axis=1, keepdims=True)
    o_ref[...] = z - jnp.log(jnp.sum(jnp.exp(z), axis=1, keepdims=True))


def _sc_chain_body(cols_ref, rows_ref, vals_ref, m0_ref, m1_ref, m2_ref,
                   m3_ref, m4_ref, zeros_ref, out_ref, pa_ref, pb_ref,
                   cols_v, rows_v, vals_v, rb0, rb1, rb2, rb3,
                   zbuf, obuf, mbuf, acc,
                   gs0, gs1, gs2, gs3, ss0, ss1, ss2, ss3):
    cid = lax.axis_index("c")
    sid = lax.axis_index("s")

    @pl.when(cid == 0)
    def _core0():
        rbase = sid * _RPT
        pltpu.sync_copy(cols_ref.at[sid], cols_v)
        pltpu.sync_copy(rows_ref.at[sid], rows_v)
        pltpu.sync_copy(vals_ref.at[sid], vals_v)
        pltpu.sync_copy(zeros_ref, zbuf)

        lane_ids = [jnp.full((16,), u, jnp.int32) for u in range(16)]

        def scale(rb, c):
            for g in range(8):
                vv = vals_v[c, pl.ds(g * 16, 16)]
                for u in range(16):
                    e = g * 16 + u
                    if e < _CHUNK:
                        rb[e, :] = rb[e, :] * jnp.take(vv, lane_ids[u])

        rbs = [rb0, rb1, rb2, rb3]
        gss = [gs0, gs1, gs2, gs3]
        sss = [ss0, ss1, ss2, ss3]

        def hop(x_src, m_src, x_dst, tag):
          with jax.named_scope(tag):
            pltpu.sync_copy(zbuf, acc.at[pl.ds(rbase, _RPT)])
            plsc.subcore_barrier()
            pltpu.async_copy(x_src.at[cols_v.at[0]], rb0, gs0)
            pltpu.async_copy(x_src.at[cols_v.at[1]], rb1, gs1)

            def body(j, _):
                for b in range(4):
                    c = 4 * j + b
                    pltpu.make_async_copy(
                        x_src.at[cols_v.at[c]], rbs[b], gss[b]).wait()
                    scale(rbs[b], c)
                    pltpu.async_copy(
                        rbs[b], acc.at[rows_v.at[c]], sss[b], add=True)
                    nb = (b + 2) % 4

                    def _wait_prev(nb=nb, pc=c - 2):
                        pltpu.make_async_copy(
                            rbs[nb], acc.at[rows_v.at[pc]], sss[nb]).wait()

                    def _issue(nb=nb, c2=c + 2):
                        pltpu.async_copy(
                            x_src.at[cols_v.at[c2]], rbs[nb], gss[nb])

                    if b < 2:
                        pl.when(j > 0)(_wait_prev)
                        _issue()
                    else:
                        _wait_prev()
                        pl.when(j < _CPT // 4 - 1)(_issue)
                return 0

            lax.fori_loop(0, _CPT // 4, body, 0)
            pltpu.make_async_copy(
                rb2, acc.at[rows_v.at[_CPT - 2]], ss2).wait()
            pltpu.make_async_copy(
                rb3, acc.at[rows_v.at[_CPT - 1]], ss3).wait()
            plsc.subcore_barrier()
          with jax.named_scope(tag + "_readout"):
            pltpu.sync_copy(acc.at[pl.ds(rbase, _RPT)], obuf)
            pltpu.sync_copy(m_src.at[pl.ds(rbase, _RPT)], mbuf)

            def radd(k, _):
                e0 = k * 5
                for u in range(5):
                    e = e0 + u
                    obuf[e, :] = obuf[e, :] + mbuf[e, :]
                return 0

            lax.fori_loop(0, _RPT // 5, radd, 0)
            pltpu.sync_copy(obuf, x_dst.at[pl.ds(rbase, _RPT)])
            plsc.subcore_barrier()

        hop(m4_ref, m3_ref, pa_ref, "hop1")
        hop(pa_ref, m2_ref, pb_ref, "hop2")
        hop(pb_ref, m1_ref, pa_ref, "hop3")
        hop(pa_ref, m0_ref, out_ref, "hop4")


def _sc_chain(cols, rows, vals, m, zeros):
    mesh = plsc.VectorSubcoreMesh(core_axis_name="c", subcore_axis_name="s")
    f = pl.kernel(
        _sc_chain_body,
        out_type=[jax.ShapeDtypeStruct((_NPAD, _NCLS), jnp.float32)] * 3,
        mesh=mesh,
        compiler_params=pltpu.CompilerParams(use_tc_tiling_on_sc=False),
        scratch_types=[
            pltpu.VMEM((_CPT, _CHUNK), jnp.int32),
            pltpu.VMEM((_CPT, _CHUNK), jnp.int32),
            pltpu.VMEM((_CPT, 128), jnp.float32),
            pltpu.VMEM((_CHUNK, _NCLS), jnp.float32),
            pltpu.VMEM((_CHUNK, _NCLS), jnp.float32),
            pltpu.VMEM((_CHUNK, _NCLS), jnp.float32),
            pltpu.VMEM((_CHUNK, _NCLS), jnp.float32),
            pltpu.VMEM((_RPT, _NCLS), jnp.float32),
            pltpu.VMEM((_RPT, _NCLS), jnp.float32),
            pltpu.VMEM((_RPT, _NCLS), jnp.float32),
            pltpu.VMEM_SHARED((_NPAD, _NCLS), jnp.float32),
        ] + [pltpu.SemaphoreType.DMA] * 8,
    )
    return f(cols, rows, vals, m[0], m[1], m[2], m[3], m[4], zeros)[0]


def kernel(features, adj_values, upper_W, upper_b, bottom_W, bottom_b,
           fc_W, fc_b, adj_indices):
    f32 = jnp.float32
    features = features.astype(f32)
    cols = adj_indices[1].astype(jnp.int32).reshape(_TILES, _CPT, _CHUNK)
    rows = adj_indices[0].astype(jnp.int32).reshape(_TILES, _CPT, _CHUNK)
    vals = jnp.pad(adj_values.astype(f32).reshape(_TILES, _CPT, _CHUNK),
                   ((0, 0), (0, 0), (0, 128 - _CHUNK)))

    w_all = jnp.transpose(upper_W, (1, 0, 2)).reshape(_D, _A1)
    b_all = upper_b.reshape(1, _A1)

    q, bias = pl.pallas_call(
        _qbias_body,
        out_shape=[
            jax.ShapeDtypeStruct((_A1, _NCLS * _NM), f32),
            jax.ShapeDtypeStruct((1, _NCLS), f32),
        ],
    )(bottom_W.astype(f32), fc_W.astype(f32), bottom_b.astype(f32),
      fc_b.astype(f32).reshape(1, _NCLS))

    nblk = _N // _ROWBLK
    m = pl.pallas_call(
        _dense_body,
        grid=(nblk,),
        in_specs=[
            pl.BlockSpec((_ROWBLK, _D), lambda i: (i, 0)),
            pl.BlockSpec((_D, _A1), lambda i: (0, 0)),
            pl.BlockSpec((1, _A1), lambda i: (0, 0)),
            pl.BlockSpec((_A1, _NCLS * _NM), lambda i: (0, 0)),
        ],
        out_specs=[pl.BlockSpec((_ROWBLK, _NCLS), lambda i: (i, 0))] * _NM,
        out_shape=[jax.ShapeDtypeStruct((_NPAD, _NCLS), f32)] * _NM,
    )(features, w_all, b_all, q)

    zeros = jnp.zeros((_RPT, _NCLS), f32)
    p = _sc_chain(cols, rows, vals, m, zeros)

    out = pl.pallas_call(
        _finish_body,
        grid=(_N // _FINBLK,),
        in_specs=[
            pl.BlockSpec((_FINBLK, _NCLS), lambda i: (i, 0)),
            pl.BlockSpec((1, _NCLS), lambda i: (0, 0)),
        ],
        out_specs=pl.BlockSpec((_FINBLK, _NCLS), lambda i: (i, 0)),
        out_shape=jax.ShapeDtypeStruct((_N, _NCLS), f32),
    )(p, bias)
    return out


# speedup vs baseline: 35.1217x; 1.4358x over previous
import functools

import jax
import jax.numpy as jnp
from jax import lax
from jax.experimental import pallas as pl
from jax.experimental.pallas import tpu as pltpu
from jax.experimental.pallas import tpu_sc as plsc

_N = 10000
_E = 320000
_D = 128
_L = 200
_A1 = 600
_NCLS = 16
_NM = 5

_CHUNK = 125
_NCHUNKS = _E // _CHUNK
_TILES = 16
_WORKERS = 32
_CPW = _NCHUNKS // _WORKERS
_HALF = 5120
_CMB = 320
_NPAD = 10240
_RPT = _NPAD // _TILES
_ROWBLK = 400
_FINBLK = 1000


def _qbias_body(bw_ref, fc_ref, bb_ref, fcb_ref, q_ref, bias_ref):
    g = [
        jnp.dot(bw_ref[i], fc_ref[_L * i:_L * (i + 1), :],
                preferred_element_type=jnp.float32)
        for i in range(3)
    ]
    zero = jnp.zeros((_L, _NCLS), jnp.float32)
    cols = []
    for k in range(_NM):
        parts = []
        for b in range(3):
            i = k - b
            parts.append(g[i][_L * b:_L * (b + 1), :] if 0 <= i <= 2 else zero)
        cols.append(jnp.concatenate(parts, axis=0))
    q_ref[...] = jnp.concatenate(cols, axis=1)
    bias = fcb_ref[...]
    for i in range(3):
        bias = bias + jnp.dot(bb_ref[i], fc_ref[_L * i:_L * (i + 1), :],
                              preferred_element_type=jnp.float32)
    bias_ref[...] = bias


def _dense_body(f_ref, w_ref, b_ref, q_ref, *m_refs):
    h = jnp.dot(f_ref[...], w_ref[...], preferred_element_type=jnp.float32)
    h = jnp.maximum(h + b_ref[...], 0.0)
    m = jnp.dot(h, q_ref[...], preferred_element_type=jnp.float32)
    for k in range(_NM):
        m_refs[k][...] = m[:, _NCLS * k:_NCLS * (k + 1)]


def _finish_body(x_ref, b_ref, o_ref):
    z = x_ref[...] + b_ref[...]
    z = z - jnp.max(z, axis=1, keepdims=True)
    o_ref[...] = z - jnp.log(jnp.sum(jnp.exp(z), axis=1, keepdims=True))


def _sc_chain_body(cols_ref, rows_ref, vals_ref, m0_ref, m1_ref, m2_ref,
                   m3_ref, m4_ref, zeros_ref,
                   out_ref, pa_ref, pb_ref, ph0_ref, ph1_ref,
                   cols_v, rows_v, vals_v, rb0, rb1, rb2, rb3,
                   zbuf, obuf, pbuf, mbuf, acc,
                   gs0, gs1, gs2, gs3, ss0, ss1, ss2, ss3, hs):
    cid = lax.axis_index("c")
    sid = lax.axis_index("s")
    w = cid * _TILES + sid
    zbase = sid * _RPT
    own = cid * _HALF + sid * _CMB
    exp = (1 - cid) * _HALF + sid * _CMB
    ps = sid * _CMB

    pltpu.sync_copy(cols_ref.at[w], cols_v)
    pltpu.sync_copy(rows_ref.at[w], rows_v)
    pltpu.sync_copy(vals_ref.at[w], vals_v)
    pltpu.sync_copy(zeros_ref, zbuf)

    lane_ids = [jnp.full((16,), u, jnp.int32) for u in range(16)]
    rbs = [rb0, rb1, rb2, rb3]
    gss = [gs0, gs1, gs2, gs3]
    sss = [ss0, ss1, ss2, ss3]

    def handshake():
        plsc.subcore_barrier()

        @pl.when(sid == 0)
        def _hs():
            pl.semaphore_signal(hs, 1, core_index=1 - cid)
            pl.semaphore_wait(hs, 1)

        plsc.subcore_barrier()

    def scale(rb, c):
        for g in range(8):
            vv = vals_v[c, pl.ds(g * 16, 16)]
            for u in range(16):
                e = g * 16 + u
                if e < _CHUNK:
                    rb[e, :] = rb[e, :] * jnp.take(vv, lane_ids[u])

    def hop(x_src, m_src, x_dst, tag, last):
      with jax.named_scope(tag):
        pltpu.sync_copy(zbuf, acc.at[pl.ds(zbase, _RPT)])
        plsc.subcore_barrier()
        pltpu.async_copy(x_src.at[cols_v.at[0]], rb0, gs0)
        pltpu.async_copy(x_src.at[cols_v.at[1]], rb1, gs1)

        def body(j, _):
            for b in range(4):
                c = 4 * j + b
                pltpu.make_async_copy(
                    x_src.at[cols_v.at[c]], rbs[b], gss[b]).wait()
                scale(rbs[b], c)
                pltpu.async_copy(
                    rbs[b], acc.at[rows_v.at[c]], sss[b], add=True)
                nb = (b + 2) % 4

                def _wait_prev(nb=nb, pc=c - 2):
                    pltpu.make_async_copy(
                        rbs[nb], acc.at[rows_v.at[pc]], sss[nb]).wait()

                def _issue(nb=nb, c2=c + 2):
                    pltpu.async_copy(
                        x_src.at[cols_v.at[c2]], rbs[nb], gss[nb])

                if b < 2:
                    pl.when(j > 0)(_wait_prev)
                    _issue()
                else:
                    _wait_prev()
                    pl.when(j < _CPW // 4 - 1)(_issue)
            return 0

        lax.fori_loop(0, _CPW // 4, body, 0)
        pltpu.make_async_copy(rb2, acc.at[rows_v.at[_CPW - 2]], ss2).wait()
        pltpu.make_async_copy(rb3, acc.at[rows_v.at[_CPW - 1]], ss3).wait()
        plsc.subcore_barrier()
      with jax.named_scope(tag + "_readout"):
        @pl.when(cid == 0)
        def _exp0():
            pltpu.sync_copy(acc.at[pl.ds(exp, _CMB)],
                            ph0_ref.at[pl.ds(ps, _CMB)])

        @pl.when(cid == 1)
        def _exp1():
            pltpu.sync_copy(acc.at[pl.ds(exp, _CMB)],
                            ph1_ref.at[pl.ds(ps, _CMB)])

        handshake()
        pltpu.sync_copy(acc.at[pl.ds(own, _CMB)], obuf)

        @pl.when(cid == 0)
        def _imp0():
            pltpu.sync_copy(ph1_ref.at[pl.ds(ps, _CMB)], pbuf)

        @pl.when(cid == 1)
        def _imp1():
            pltpu.sync_copy(ph0_ref.at[pl.ds(ps, _CMB)], pbuf)

        pltpu.sync_copy(m_src.at[pl.ds(own, _CMB)], mbuf)

        def radd(k, _):
            e0 = k * 5
            for u in range(5):
                e = e0 + u
                obuf[e, :] = (obuf[e, :] + pbuf[e, :]) + mbuf[e, :]
            return 0

        lax.fori_loop(0, _CMB // 5, radd, 0)
        pltpu.sync_copy(obuf, x_dst.at[pl.ds(own, _CMB)])
        if not last:
            handshake()

    hop(m4_ref, m3_ref, pa_ref, "hop1", False)
    hop(pa_ref, m2_ref, pb_ref, "hop2", False)
    hop(pb_ref, m1_ref, pa_ref, "hop3", False)
    hop(pa_ref, m0_ref, out_ref, "hop4", True)


def _sc_chain(cols, rows, vals, m, zeros):
    mesh = plsc.VectorSubcoreMesh(core_axis_name="c", subcore_axis_name="s")
    f = pl.kernel(
        _sc_chain_body,
        out_type=[jax.ShapeDtypeStruct((_NPAD, _NCLS), jnp.float32)] * 3
        + [jax.ShapeDtypeStruct((_HALF, _NCLS), jnp.float32)] * 2,
        mesh=mesh,
        compiler_params=pltpu.CompilerParams(use_tc_tiling_on_sc=False),
        scratch_types=[
            pltpu.VMEM((_CPW, _CHUNK), jnp.int32),
            pltpu.VMEM((_CPW, _CHUNK), jnp.int32),
            pltpu.VMEM((_CPW, 128), jnp.float32),
            pltpu.VMEM((_CHUNK, _NCLS), jnp.float32),
            pltpu.VMEM((_CHUNK, _NCLS), jnp.float32),
            pltpu.VMEM((_CHUNK, _NCLS), jnp.float32),
            pltpu.VMEM((_CHUNK, _NCLS), jnp.float32),
            pltpu.VMEM((_RPT, _NCLS), jnp.float32),
            pltpu.VMEM((_CMB, _NCLS), jnp.float32),
            pltpu.VMEM((_CMB, _NCLS), jnp.float32),
            pltpu.VMEM((_CMB, _NCLS), jnp.float32),
            pltpu.VMEM_SHARED((_NPAD, _NCLS), jnp.float32),
        ] + [pltpu.SemaphoreType.DMA] * 8
        + [pltpu.SemaphoreType.REGULAR],
    )
    return f(cols, rows, vals, m[0], m[1], m[2], m[3], m[4], zeros)[0]


def kernel(features, adj_values, upper_W, upper_b, bottom_W, bottom_b,
           fc_W, fc_b, adj_indices):
    f32 = jnp.float32
    features = features.astype(f32)
    cols = adj_indices[1].astype(jnp.int32).reshape(_WORKERS, _CPW, _CHUNK)
    rows = adj_indices[0].astype(jnp.int32).reshape(_WORKERS, _CPW, _CHUNK)
    vals = jnp.pad(adj_values.astype(f32).reshape(_WORKERS, _CPW, _CHUNK),
                   ((0, 0), (0, 0), (0, 128 - _CHUNK)))

    w_all = jnp.transpose(upper_W, (1, 0, 2)).reshape(_D, _A1)
    b_all = upper_b.reshape(1, _A1)

    q, bias = pl.pallas_call(
        _qbias_body,
        out_shape=[
            jax.ShapeDtypeStruct((_A1, _NCLS * _NM), f32),
            jax.ShapeDtypeStruct((1, _NCLS), f32),
        ],
    )(bottom_W.astype(f32), fc_W.astype(f32), bottom_b.astype(f32),
      fc_b.astype(f32).reshape(1, _NCLS))

    nblk = _N // _ROWBLK
    m = pl.pallas_call(
        _dense_body,
        grid=(nblk,),
        in_specs=[
            pl.BlockSpec((_ROWBLK, _D), lambda i: (i, 0)),
            pl.BlockSpec((_D, _A1), lambda i: (0, 0)),
            pl.BlockSpec((1, _A1), lambda i: (0, 0)),
            pl.BlockSpec((_A1, _NCLS * _NM), lambda i: (0, 0)),
        ],
        out_specs=[pl.BlockSpec((_ROWBLK, _NCLS), lambda i: (i, 0))] * _NM,
        out_shape=[jax.ShapeDtypeStruct((_NPAD, _NCLS), f32)] * _NM,
    )(features, w_all, b_all, q)

    zeros = jnp.zeros((_RPT, _NCLS), f32)
    p = _sc_chain(cols, rows, vals, m, zeros)

    out = pl.pallas_call(
        _finish_body,
        grid=(_N // _FINBLK,),
        in_specs=[
            pl.BlockSpec((_FINBLK, _NCLS), lambda i: (i, 0)),
            pl.BlockSpec((1, _NCLS), lambda i: (0, 0)),
        ],
        out_specs=pl.BlockSpec((_FINBLK, _NCLS), lambda i: (i, 0)),
        out_shape=jax.ShapeDtypeStruct((_N, _NCLS), f32),
    )(p, bias)
    return out


# speedup vs baseline: 48.4328x; 1.3790x over previous
import functools

import jax
import jax.numpy as jnp
from jax import lax
from jax.experimental import pallas as pl
from jax.experimental.pallas import tpu as pltpu
from jax.experimental.pallas import tpu_sc as plsc

_N = 10000
_E = 320000
_D = 128
_L = 200
_A1 = 600
_NCLS = 16
_NM = 5

_CHUNK = 125
_NCHUNKS = _E // _CHUNK
_TILES = 16
_WORKERS = 32
_CPW = _NCHUNKS // _WORKERS
_HALF = 5120
_CMB = 320
_NPAD = 10240
_RPT = _NPAD // _TILES
_ROWBLK = 400
_FINBLK = 1000


def _qbias_body(bw_ref, fc_ref, bb_ref, fcb_ref, q_ref, bias_ref):
    g = [
        jnp.dot(bw_ref[i], fc_ref[_L * i:_L * (i + 1), :],
                preferred_element_type=jnp.float32)
        for i in range(3)
    ]
    zero = jnp.zeros((_L, _NCLS), jnp.float32)
    cols = []
    for k in range(_NM):
        parts = []
        for b in range(3):
            i = k - b
            parts.append(g[i][_L * b:_L * (b + 1), :] if 0 <= i <= 2 else zero)
        cols.append(jnp.concatenate(parts, axis=0))
    q_ref[...] = jnp.concatenate(cols, axis=1)
    bias = fcb_ref[...]
    for i in range(3):
        bias = bias + jnp.dot(bb_ref[i], fc_ref[_L * i:_L * (i + 1), :],
                              preferred_element_type=jnp.float32)
    bias_ref[...] = bias


def _dense_body(f_ref, w_ref, b_ref, q_ref, *m_refs):
    h = jnp.dot(f_ref[...], w_ref[...], preferred_element_type=jnp.float32)
    h = jnp.maximum(h + b_ref[...], 0.0)
    m = jnp.dot(h, q_ref[...], preferred_element_type=jnp.float32)
    for k in range(_NM):
        m_refs[k][...] = m[:, _NCLS * k:_NCLS * (k + 1)]


def _finish_body(x_ref, b_ref, o_ref):
    z = x_ref[...] + b_ref[...]
    z = z - jnp.max(z, axis=1, keepdims=True)
    o_ref[...] = z - jnp.log(jnp.sum(jnp.exp(z), axis=1, keepdims=True))


def _sc_chain_body(cols_ref, rows_ref, vals_ref, m0_ref, m1_ref, m2_ref,
                   m3_ref, m4_ref, zeros_ref,
                   out_ref, pa_ref, pb_ref, ph0_ref, ph1_ref,
                   cols_v, rows_v, vals_v, rb0, rb1, rb2, rb3,
                   zbuf, xbuf, obuf, pbuf, mbuf, acc, xloc,
                   gs0, gs1, gs2, gs3, ss0, ss1, ss2, ss3, hs):
    cid = lax.axis_index("c")
    sid = lax.axis_index("s")
    w = cid * _TILES + sid
    zbase = sid * _RPT
    own = cid * _HALF + sid * _CMB
    exp = (1 - cid) * _HALF + sid * _CMB
    ps = sid * _CMB

    pltpu.sync_copy(cols_ref.at[w], cols_v)
    pltpu.sync_copy(rows_ref.at[w], rows_v)
    pltpu.sync_copy(vals_ref.at[w], vals_v)
    pltpu.sync_copy(zeros_ref, zbuf)
    pltpu.sync_copy(m4_ref.at[pl.ds(sid * _RPT, _RPT)], xbuf)
    pltpu.sync_copy(xbuf, xloc.at[pl.ds(sid * _RPT, _RPT)])

    lane_ids = [jnp.full((16,), u, jnp.int32) for u in range(16)]
    rbs = [rb0, rb1, rb2, rb3]
    gss = [gs0, gs1, gs2, gs3]
    sss = [ss0, ss1, ss2, ss3]

    def handshake():
        plsc.subcore_barrier()

        @pl.when(sid == 0)
        def _hs():
            pl.semaphore_signal(hs, 1, core_index=1 - cid)
            pl.semaphore_wait(hs, 1)

        plsc.subcore_barrier()

    def scale(rb, c):
        for g in range(8):
            vv = vals_v[c, pl.ds(g * 16, 16)]
            for u in range(16):
                e = g * 16 + u
                if e < _CHUNK:
                    rb[e, :] = rb[e, :] * jnp.take(vv, lane_ids[u])

    def hop(m_src, x_dst, tag, last):
      with jax.named_scope(tag):
        pltpu.sync_copy(zbuf, acc.at[pl.ds(zbase, _RPT)])
        plsc.subcore_barrier()
        pltpu.async_copy(xloc.at[cols_v.at[0]], rb0, gs0)
        pltpu.async_copy(xloc.at[cols_v.at[1]], rb1, gs1)

        def body(j, _):
            for b in range(4):
                c = 4 * j + b
                pltpu.make_async_copy(
                    xloc.at[cols_v.at[c]], rbs[b], gss[b]).wait()
                scale(rbs[b], c)
                pltpu.async_copy(
                    rbs[b], acc.at[rows_v.at[c]], sss[b], add=True)
                nb = (b + 2) % 4

                def _wait_prev(nb=nb, pc=c - 2):
                    pltpu.make_async_copy(
                        rbs[nb], acc.at[rows_v.at[pc]], sss[nb]).wait()

                def _issue(nb=nb, c2=c + 2):
                    pltpu.async_copy(
                        xloc.at[cols_v.at[c2]], rbs[nb], gss[nb])

                if b < 2:
                    pl.when(j > 0)(_wait_prev)
                    _issue()
                else:
                    _wait_prev()
                    pl.when(j < _CPW // 4 - 1)(_issue)
            return 0

        lax.fori_loop(0, _CPW // 4, body, 0)
        pltpu.make_async_copy(rb2, acc.at[rows_v.at[_CPW - 2]], ss2).wait()
        pltpu.make_async_copy(rb3, acc.at[rows_v.at[_CPW - 1]], ss3).wait()
        plsc.subcore_barrier()
      with jax.named_scope(tag + "_readout"):
        @pl.when(cid == 0)
        def _exp0():
            pltpu.sync_copy(acc.at[pl.ds(exp, _CMB)],
                            ph0_ref.at[pl.ds(ps, _CMB)])

        @pl.when(cid == 1)
        def _exp1():
            pltpu.sync_copy(acc.at[pl.ds(exp, _CMB)],
                            ph1_ref.at[pl.ds(ps, _CMB)])

        handshake()
        pltpu.sync_copy(acc.at[pl.ds(own, _CMB)], obuf)

        @pl.when(cid == 0)
        def _imp0():
            pltpu.sync_copy(ph1_ref.at[pl.ds(ps, _CMB)], pbuf)

        @pl.when(cid == 1)
        def _imp1():
            pltpu.sync_copy(ph0_ref.at[pl.ds(ps, _CMB)], pbuf)

        pltpu.sync_copy(m_src.at[pl.ds(own, _CMB)], mbuf)

        def radd(k, _):
            e0 = k * 5
            for u in range(5):
                e = e0 + u
                obuf[e, :] = (obuf[e, :] + pbuf[e, :]) + mbuf[e, :]
            return 0

        lax.fori_loop(0, _CMB // 5, radd, 0)
        pltpu.sync_copy(obuf, x_dst.at[pl.ds(own, _CMB)])
        if not last:
            pltpu.sync_copy(obuf, xloc.at[pl.ds(own, _CMB)])
            handshake()
            pltpu.sync_copy(x_dst.at[pl.ds(exp, _CMB)], pbuf)
            pltpu.sync_copy(pbuf, xloc.at[pl.ds(exp, _CMB)])

    hop(m3_ref, pa_ref, "hop1", False)
    hop(m2_ref, pb_ref, "hop2", False)
    hop(m1_ref, pa_ref, "hop3", False)
    hop(m0_ref, out_ref, "hop4", True)


def _sc_chain(cols, rows, vals, m, zeros):
    mesh = plsc.VectorSubcoreMesh(core_axis_name="c", subcore_axis_name="s")
    f = pl.kernel(
        _sc_chain_body,
        out_type=[jax.ShapeDtypeStruct((_NPAD, _NCLS), jnp.float32)] * 3
        + [jax.ShapeDtypeStruct((_HALF, _NCLS), jnp.float32)] * 2,
        mesh=mesh,
        compiler_params=pltpu.CompilerParams(use_tc_tiling_on_sc=False),
        scratch_types=[
            pltpu.VMEM((_CPW, _CHUNK), jnp.int32),
            pltpu.VMEM((_CPW, _CHUNK), jnp.int32),
            pltpu.VMEM((_CPW, 128), jnp.float32),
            pltpu.VMEM((_CHUNK, _NCLS), jnp.float32),
            pltpu.VMEM((_CHUNK, _NCLS), jnp.float32),
            pltpu.VMEM((_CHUNK, _NCLS), jnp.float32),
            pltpu.VMEM((_CHUNK, _NCLS), jnp.float32),
            pltpu.VMEM((_RPT, _NCLS), jnp.float32),
            pltpu.VMEM((_RPT, _NCLS), jnp.float32),
            pltpu.VMEM((_CMB, _NCLS), jnp.float32),
            pltpu.VMEM((_CMB, _NCLS), jnp.float32),
            pltpu.VMEM((_CMB, _NCLS), jnp.float32),
            pltpu.VMEM_SHARED((_NPAD, _NCLS), jnp.float32),
            pltpu.VMEM_SHARED((_NPAD, _NCLS), jnp.float32),
        ] + [pltpu.SemaphoreType.DMA] * 8
        + [pltpu.SemaphoreType.REGULAR],
    )
    return f(cols, rows, vals, m[0], m[1], m[2], m[3], m[4], zeros)[0]


def kernel(features, adj_values, upper_W, upper_b, bottom_W, bottom_b,
           fc_W, fc_b, adj_indices):
    f32 = jnp.float32
    features = features.astype(f32)
    cols = adj_indices[1].astype(jnp.int32).reshape(_WORKERS, _CPW, _CHUNK)
    rows = adj_indices[0].astype(jnp.int32).reshape(_WORKERS, _CPW, _CHUNK)
    vals = jnp.pad(adj_values.astype(f32).reshape(_WORKERS, _CPW, _CHUNK),
                   ((0, 0), (0, 0), (0, 128 - _CHUNK)))

    w_all = jnp.transpose(upper_W, (1, 0, 2)).reshape(_D, _A1)
    b_all = upper_b.reshape(1, _A1)

    q, bias = pl.pallas_call(
        _qbias_body,
        out_shape=[
            jax.ShapeDtypeStruct((_A1, _NCLS * _NM), f32),
            jax.ShapeDtypeStruct((1, _NCLS), f32),
        ],
    )(bottom_W.astype(f32), fc_W.astype(f32), bottom_b.astype(f32),
      fc_b.astype(f32).reshape(1, _NCLS))

    nblk = _N // _ROWBLK
    m = pl.pallas_call(
        _dense_body,
        grid=(nblk,),
        in_specs=[
            pl.BlockSpec((_ROWBLK, _D), lambda i: (i, 0)),
            pl.BlockSpec((_D, _A1), lambda i: (0, 0)),
            pl.BlockSpec((1, _A1), lambda i: (0, 0)),
            pl.BlockSpec((_A1, _NCLS * _NM), lambda i: (0, 0)),
        ],
        out_specs=[pl.BlockSpec((_ROWBLK, _NCLS), lambda i: (i, 0))] * _NM,
        out_shape=[jax.ShapeDtypeStruct((_NPAD, _NCLS), f32)] * _NM,
    )(features, w_all, b_all, q)

    zeros = jnp.zeros((_RPT, _NCLS), f32)
    p = _sc_chain(cols, rows, vals, m, zeros)

    out = pl.pallas_call(
        _finish_body,
        grid=(_N // _FINBLK,),
        in_specs=[
            pl.BlockSpec((_FINBLK, _NCLS), lambda i: (i, 0)),
            pl.BlockSpec((1, _NCLS), lambda i: (0, 0)),
        ],
        out_specs=pl.BlockSpec((_FINBLK, _NCLS), lambda i: (i, 0)),
        out_shape=jax.ShapeDtypeStruct((_N, _NCLS), f32),
    )(p, bias)
    return out


# speedup vs baseline: 50.7537x; 1.0479x over previous
import functools

import jax
import jax.numpy as jnp
from jax import lax
from jax.experimental import pallas as pl
from jax.experimental.pallas import tpu as pltpu
from jax.experimental.pallas import tpu_sc as plsc

_N = 10000
_E = 320000
_D = 128
_L = 200
_A1 = 600
_NCLS = 16
_NM = 5

_CHUNK = 125
_NCHUNKS = _E // _CHUNK
_TILES = 16
_WORKERS = 32
_CPW = _NCHUNKS // _WORKERS
_HALF = 5120
_CMB = 320
_NPAD = 10240
_RPT = _NPAD // _TILES
_ROWBLK = 2000
_FINBLK = 2000


def _qbias_body(bw_ref, fc_ref, bb_ref, fcb_ref, q_ref, bias_ref):
    g = [
        jnp.dot(bw_ref[i], fc_ref[_L * i:_L * (i + 1), :],
                preferred_element_type=jnp.float32)
        for i in range(3)
    ]
    zero = jnp.zeros((_L, _NCLS), jnp.float32)
    cols = []
    for k in range(_NM):
        parts = []
        for b in range(3):
            i = k - b
            parts.append(g[i][_L * b:_L * (b + 1), :] if 0 <= i <= 2 else zero)
        cols.append(jnp.concatenate(parts, axis=0))
    q_ref[...] = jnp.concatenate(cols, axis=1)
    bias = fcb_ref[...]
    for i in range(3):
        bias = bias + jnp.dot(bb_ref[i], fc_ref[_L * i:_L * (i + 1), :],
                              preferred_element_type=jnp.float32)
    bias_ref[...] = bias


def _dense_body(f_ref, w_ref, b_ref, q_ref, *m_refs):
    h = jnp.dot(f_ref[...].astype(jnp.bfloat16), w_ref[...].astype(jnp.bfloat16),
                preferred_element_type=jnp.float32)
    h = jnp.maximum(h + b_ref[...], 0.0)
    m = jnp.dot(h.astype(jnp.bfloat16), q_ref[...].astype(jnp.bfloat16),
                preferred_element_type=jnp.float32)
    for k in range(_NM):
        m_refs[k][...] = m[:, _NCLS * k:_NCLS * (k + 1)]


def _finish_body(x_ref, b_ref, o_ref):
    z = x_ref[...] + b_ref[...]
    z = z - jnp.max(z, axis=1, keepdims=True)
    o_ref[...] = z - jnp.log(jnp.sum(jnp.exp(z), axis=1, keepdims=True))


def _sc_chain_body(cols_ref, rows_ref, vals_ref, m0_ref, m1_ref, m2_ref,
                   m3_ref, m4_ref,
                   out_ref, pa_ref, pb_ref, ph0_ref, ph1_ref,
                   cols_v, rows_v, vals_v, rb0, rb1, rb2, rb3,
                   zbuf, xbuf, obuf, pbuf, mbuf, acc, xloc,
                   gs0, gs1, gs2, gs3, ss0, ss1, ss2, ss3, hs):
    cid = lax.axis_index("c")
    sid = lax.axis_index("s")
    w = cid * _TILES + sid
    zbase = sid * _RPT
    own = cid * _HALF + sid * _CMB
    exp = (1 - cid) * _HALF + sid * _CMB
    ps = sid * _CMB

    pltpu.sync_copy(cols_ref.at[w], cols_v)
    pltpu.sync_copy(rows_ref.at[w], rows_v)
    pltpu.sync_copy(vals_ref.at[w], vals_v)

    def _zfill(i, _):
        zbuf[i, :] = jnp.zeros((16,), jnp.float32)
        return 0

    lax.fori_loop(0, _RPT, _zfill, 0)
    pltpu.sync_copy(m4_ref.at[pl.ds(sid * _RPT, _RPT)], xbuf)
    pltpu.sync_copy(xbuf, xloc.at[pl.ds(sid * _RPT, _RPT)])

    lane_ids = [jnp.full((16,), u, jnp.int32) for u in range(16)]
    rbs = [rb0, rb1, rb2, rb3]
    gss = [gs0, gs1, gs2, gs3]
    sss = [ss0, ss1, ss2, ss3]

    def handshake():
        plsc.subcore_barrier()

        @pl.when(sid == 0)
        def _hs():
            pl.semaphore_signal(hs, 1, core_index=1 - cid)
            pl.semaphore_wait(hs, 1)

        plsc.subcore_barrier()

    def scale(rb, c):
        for g in range(8):
            vv = vals_v[c, pl.ds(g * 16, 16)]
            for u in range(16):
                e = g * 16 + u
                if e < _CHUNK:
                    rb[e, :] = rb[e, :] * jnp.take(vv, lane_ids[u])

    def hop(m_src, x_dst, tag, last):
      with jax.named_scope(tag):
        pltpu.sync_copy(zbuf, acc.at[pl.ds(zbase, _RPT)])
        plsc.subcore_barrier()
        pltpu.async_copy(xloc.at[cols_v.at[0]], rb0, gs0)
        pltpu.async_copy(xloc.at[cols_v.at[1]], rb1, gs1)

        def body(j, _):
            for b in range(4):
                c = 4 * j + b
                pltpu.make_async_copy(
                    xloc.at[cols_v.at[c]], rbs[b], gss[b]).wait()
                scale(rbs[b], c)
                pltpu.async_copy(
                    rbs[b], acc.at[rows_v.at[c]], sss[b], add=True)
                nb = (b + 2) % 4

                def _wait_prev(nb=nb, pc=c - 2):
                    pltpu.make_async_copy(
                        rbs[nb], acc.at[rows_v.at[pc]], sss[nb]).wait()

                def _issue(nb=nb, c2=c + 2):
                    pltpu.async_copy(
                        xloc.at[cols_v.at[c2]], rbs[nb], gss[nb])

                if b < 2:
                    pl.when(j > 0)(_wait_prev)
                    _issue()
                else:
                    _wait_prev()
                    pl.when(j < _CPW // 4 - 1)(_issue)
            return 0

        lax.fori_loop(0, _CPW // 4, body, 0)
        pltpu.make_async_copy(rb2, acc.at[rows_v.at[_CPW - 2]], ss2).wait()
        pltpu.make_async_copy(rb3, acc.at[rows_v.at[_CPW - 1]], ss3).wait()
        plsc.subcore_barrier()
      with jax.named_scope(tag + "_readout"):
        @pl.when(cid == 0)
        def _exp0():
            pltpu.sync_copy(acc.at[pl.ds(exp, _CMB)],
                            ph0_ref.at[pl.ds(ps, _CMB)])

        @pl.when(cid == 1)
        def _exp1():
            pltpu.sync_copy(acc.at[pl.ds(exp, _CMB)],
                            ph1_ref.at[pl.ds(ps, _CMB)])

        handshake()
        pltpu.sync_copy(acc.at[pl.ds(own, _CMB)], obuf)

        @pl.when(cid == 0)
        def _imp0():
            pltpu.sync_copy(ph1_ref.at[pl.ds(ps, _CMB)], pbuf)

        @pl.when(cid == 1)
        def _imp1():
            pltpu.sync_copy(ph0_ref.at[pl.ds(ps, _CMB)], pbuf)

        pltpu.sync_copy(m_src.at[pl.ds(own, _CMB)], mbuf)

        def radd(k, _):
            e0 = k * 5
            for u in range(5):
                e = e0 + u
                obuf[e, :] = (obuf[e, :] + pbuf[e, :]) + mbuf[e, :]
            return 0

        lax.fori_loop(0, _CMB // 5, radd, 0)
        pltpu.sync_copy(obuf, x_dst.at[pl.ds(own, _CMB)])
        if not last:
            pltpu.sync_copy(obuf, xloc.at[pl.ds(own, _CMB)])
            handshake()
            pltpu.sync_copy(x_dst.at[pl.ds(exp, _CMB)], pbuf)
            pltpu.sync_copy(pbuf, xloc.at[pl.ds(exp, _CMB)])

    hop(m3_ref, pa_ref, "hop1", False)
    hop(m2_ref, pb_ref, "hop2", False)
    hop(m1_ref, pa_ref, "hop3", False)
    hop(m0_ref, out_ref, "hop4", True)


def _sc_chain(cols, rows, vals, m):
    mesh = plsc.VectorSubcoreMesh(core_axis_name="c", subcore_axis_name="s")
    f = pl.kernel(
        _sc_chain_body,
        out_type=[jax.ShapeDtypeStruct((_NPAD, _NCLS), jnp.float32)] * 3
        + [jax.ShapeDtypeStruct((_HALF, _NCLS), jnp.float32)] * 2,
        mesh=mesh,
        compiler_params=pltpu.CompilerParams(use_tc_tiling_on_sc=False),
        scratch_types=[
            pltpu.VMEM((_CPW, _CHUNK), jnp.int32),
            pltpu.VMEM((_CPW, _CHUNK), jnp.int32),
            pltpu.VMEM((_CPW, 128), jnp.float32),
            pltpu.VMEM((_CHUNK, _NCLS), jnp.float32),
            pltpu.VMEM((_CHUNK, _NCLS), jnp.float32),
            pltpu.VMEM((_CHUNK, _NCLS), jnp.float32),
            pltpu.VMEM((_CHUNK, _NCLS), jnp.float32),
            pltpu.VMEM((_RPT, _NCLS), jnp.float32),
            pltpu.VMEM((_RPT, _NCLS), jnp.float32),
            pltpu.VMEM((_CMB, _NCLS), jnp.float32),
            pltpu.VMEM((_CMB, _NCLS), jnp.float32),
            pltpu.VMEM((_CMB, _NCLS), jnp.float32),
            pltpu.VMEM_SHARED((_NPAD, _NCLS), jnp.float32),
            pltpu.VMEM_SHARED((_NPAD, _NCLS), jnp.float32),
        ] + [pltpu.SemaphoreType.DMA] * 8
        + [pltpu.SemaphoreType.REGULAR],
    )
    return f(cols, rows, vals, m[0], m[1], m[2], m[3], m[4])[0]


def kernel(features, adj_values, upper_W, upper_b, bottom_W, bottom_b,
           fc_W, fc_b, adj_indices):
    f32 = jnp.float32
    features = features.astype(f32)
    cols = adj_indices[1].astype(jnp.int32).reshape(_WORKERS, _CPW, _CHUNK)
    rows = adj_indices[0].astype(jnp.int32).reshape(_WORKERS, _CPW, _CHUNK)
    vals = jnp.pad(adj_values.astype(f32).reshape(_WORKERS, _CPW, _CHUNK),
                   ((0, 0), (0, 0), (0, 128 - _CHUNK)))

    w_all = jnp.transpose(upper_W, (1, 0, 2)).reshape(_D, _A1)
    b_all = upper_b.reshape(1, _A1)

    q, bias = pl.pallas_call(
        _qbias_body,
        out_shape=[
            jax.ShapeDtypeStruct((_A1, _NCLS * _NM), f32),
            jax.ShapeDtypeStruct((1, _NCLS), f32),
        ],
    )(bottom_W.astype(f32), fc_W.astype(f32), bottom_b.astype(f32),
      fc_b.astype(f32).reshape(1, _NCLS))

    nblk = _N // _ROWBLK
    m = pl.pallas_call(
        _dense_body,
        grid=(nblk,),
        in_specs=[
            pl.BlockSpec((_ROWBLK, _D), lambda i: (i, 0)),
            pl.BlockSpec((_D, _A1), lambda i: (0, 0)),
            pl.BlockSpec((1, _A1), lambda i: (0, 0)),
            pl.BlockSpec((_A1, _NCLS * _NM), lambda i: (0, 0)),
        ],
        out_specs=[pl.BlockSpec((_ROWBLK, _NCLS), lambda i: (i, 0))] * _NM,
        out_shape=[jax.ShapeDtypeStruct((_NPAD, _NCLS), f32)] * _NM,
    )(features, w_all, b_all, q)

    p = _sc_chain(cols, rows, vals, m)

    out = pl.pallas_call(
        _finish_body,
        grid=(_N // _FINBLK,),
        in_specs=[
            pl.BlockSpec((_FINBLK, _NCLS), lambda i: (i, 0)),
            pl.BlockSpec((1, _NCLS), lambda i: (0, 0)),
        ],
        out_specs=pl.BlockSpec((_FINBLK, _NCLS), lambda i: (i, 0)),
        out_shape=jax.ShapeDtypeStruct((_N, _NCLS), f32),
    )(p, bias)
    return out


# speedup vs baseline: 51.7944x; 1.0205x over previous
import functools

import jax
import jax.numpy as jnp
from jax import lax
from jax.experimental import pallas as pl
from jax.experimental.pallas import tpu as pltpu
from jax.experimental.pallas import tpu_sc as plsc

_N = 10000
_E = 320000
_D = 128
_L = 200
_A1 = 600
_NCLS = 16
_NM = 5

_CHUNK = 125
_NCHUNKS = _E // _CHUNK
_TILES = 16
_WORKERS = 32
_CPW = _NCHUNKS // _WORKERS
_HALF = 5120
_CMB = 320
_NPAD = 10240
_RPT = _NPAD // _TILES
_ROWBLK = 2000
_FINBLK = 2000


def _make_q(bw_ref, fc_ref):
    g = [
        jnp.dot(bw_ref[i], fc_ref[_L * i:_L * (i + 1), :],
                preferred_element_type=jnp.float32)
        for i in range(3)
    ]
    zero = jnp.zeros((_L, _NCLS), jnp.float32)
    cols = []
    for k in range(_NM):
        parts = []
        for b in range(3):
            i = k - b
            parts.append(g[i][_L * b:_L * (b + 1), :] if 0 <= i <= 2 else zero)
        cols.append(jnp.concatenate(parts, axis=0))
    return jnp.concatenate(cols, axis=1)


def _dense_body(f_ref, w_ref, b_ref, bw_ref, fc_ref, *m_refs):
    q = _make_q(bw_ref, fc_ref)
    h = jnp.dot(f_ref[...].astype(jnp.bfloat16), w_ref[...].astype(jnp.bfloat16),
                preferred_element_type=jnp.float32)
    h = jnp.maximum(h + b_ref[...], 0.0)
    m = jnp.dot(h.astype(jnp.bfloat16), q.astype(jnp.bfloat16),
                preferred_element_type=jnp.float32)
    for k in range(_NM):
        m_refs[k][...] = m[:, _NCLS * k:_NCLS * (k + 1)]


def _finish_body(x_ref, bb_ref, fc_ref, fcb_ref, o_ref):
    bias = fcb_ref[...]
    for i in range(3):
        bias = bias + jnp.dot(bb_ref[i], fc_ref[_L * i:_L * (i + 1), :],
                              preferred_element_type=jnp.float32)
    z = x_ref[...] + bias
    z = z - jnp.max(z, axis=1, keepdims=True)
    o_ref[...] = z - jnp.log(jnp.sum(jnp.exp(z), axis=1, keepdims=True))


def _sc_chain_body(cols_ref, rows_ref, vals_ref, m0_ref, m1_ref, m2_ref,
                   m3_ref, m4_ref,
                   out_ref, pa_ref, pb_ref, ph0_ref, ph1_ref,
                   cols_v, rows_v, vals_v, rb0, rb1, rb2, rb3,
                   zbuf, xbuf, obuf, pbuf, mbuf, acc, xloc,
                   gs0, gs1, gs2, gs3, ss0, ss1, ss2, ss3, hs):
    cid = lax.axis_index("c")
    sid = lax.axis_index("s")
    w = cid * _TILES + sid
    zbase = sid * _RPT
    own = cid * _HALF + sid * _CMB
    exp = (1 - cid) * _HALF + sid * _CMB
    ps = sid * _CMB

    pltpu.sync_copy(cols_ref.at[w], cols_v)
    pltpu.sync_copy(rows_ref.at[w], rows_v)
    pltpu.sync_copy(vals_ref.at[w], vals_v.at[pl.ds(0, _CPW * _CHUNK)])

    def _zfill(i, _):
        zbuf[i, :] = jnp.zeros((16,), jnp.float32)
        return 0

    lax.fori_loop(0, _RPT, _zfill, 0)
    pltpu.sync_copy(m4_ref.at[pl.ds(sid * _RPT, _RPT)], xbuf)
    pltpu.sync_copy(xbuf, xloc.at[pl.ds(sid * _RPT, _RPT)])

    lane_ids = [jnp.full((16,), u, jnp.int32) for u in range(16)]
    rbs = [rb0, rb1, rb2, rb3]
    gss = [gs0, gs1, gs2, gs3]
    sss = [ss0, ss1, ss2, ss3]

    def handshake():
        plsc.subcore_barrier()

        @pl.when(sid == 0)
        def _hs():
            pl.semaphore_signal(hs, 1, core_index=1 - cid)
            pl.semaphore_wait(hs, 1)

        plsc.subcore_barrier()

    def scale(rb, c):
        for g in range(8):
            vv = vals_v[pl.ds(c * _CHUNK + g * 16, 16)]
            for u in range(16):
                e = g * 16 + u
                if e < _CHUNK:
                    rb[e, :] = rb[e, :] * jnp.take(vv, lane_ids[u])

    def hop(m_src, x_dst, tag, last):
      with jax.named_scope(tag):
        pltpu.sync_copy(zbuf, acc.at[pl.ds(zbase, _RPT)])
        plsc.subcore_barrier()
        pltpu.async_copy(xloc.at[cols_v.at[0]], rb0, gs0)
        pltpu.async_copy(xloc.at[cols_v.at[1]], rb1, gs1)

        def body(j, _):
            for b in range(4):
                c = 4 * j + b
                pltpu.make_async_copy(
                    xloc.at[cols_v.at[c]], rbs[b], gss[b]).wait()
                scale(rbs[b], c)
                pltpu.async_copy(
                    rbs[b], acc.at[rows_v.at[c]], sss[b], add=True)
                nb = (b + 2) % 4

                def _wait_prev(nb=nb, pc=c - 2):
                    pltpu.make_async_copy(
                        rbs[nb], acc.at[rows_v.at[pc]], sss[nb]).wait()

                def _issue(nb=nb, c2=c + 2):
                    pltpu.async_copy(
                        xloc.at[cols_v.at[c2]], rbs[nb], gss[nb])

                if b < 2:
                    pl.when(j > 0)(_wait_prev)
                    _issue()
                else:
                    _wait_prev()
                    pl.when(j < _CPW // 4 - 1)(_issue)
            return 0

        lax.fori_loop(0, _CPW // 4, body, 0)
        pltpu.make_async_copy(rb2, acc.at[rows_v.at[_CPW - 2]], ss2).wait()
        pltpu.make_async_copy(rb3, acc.at[rows_v.at[_CPW - 1]], ss3).wait()
        plsc.subcore_barrier()
      with jax.named_scope(tag + "_readout"):
        @pl.when(cid == 0)
        def _exp0():
            pltpu.sync_copy(acc.at[pl.ds(exp, _CMB)],
                            ph0_ref.at[pl.ds(ps, _CMB)])

        @pl.when(cid == 1)
        def _exp1():
            pltpu.sync_copy(acc.at[pl.ds(exp, _CMB)],
                            ph1_ref.at[pl.ds(ps, _CMB)])

        handshake()
        pltpu.sync_copy(acc.at[pl.ds(own, _CMB)], obuf)

        @pl.when(cid == 0)
        def _imp0():
            pltpu.sync_copy(ph1_ref.at[pl.ds(ps, _CMB)], pbuf)

        @pl.when(cid == 1)
        def _imp1():
            pltpu.sync_copy(ph0_ref.at[pl.ds(ps, _CMB)], pbuf)

        pltpu.sync_copy(m_src.at[pl.ds(own, _CMB)], mbuf)

        def radd(k, _):
            e0 = k * 5
            for u in range(5):
                e = e0 + u
                obuf[e, :] = (obuf[e, :] + pbuf[e, :]) + mbuf[e, :]
            return 0

        lax.fori_loop(0, _CMB // 5, radd, 0)
        pltpu.sync_copy(obuf, x_dst.at[pl.ds(own, _CMB)])
        if not last:
            pltpu.sync_copy(obuf, xloc.at[pl.ds(own, _CMB)])
            handshake()
            pltpu.sync_copy(x_dst.at[pl.ds(exp, _CMB)], pbuf)
            pltpu.sync_copy(pbuf, xloc.at[pl.ds(exp, _CMB)])

    hop(m3_ref, pa_ref, "hop1", False)
    hop(m2_ref, pb_ref, "hop2", False)
    hop(m1_ref, pa_ref, "hop3", False)
    hop(m0_ref, out_ref, "hop4", True)


def _sc_chain(cols, rows, vals, m):
    mesh = plsc.VectorSubcoreMesh(core_axis_name="c", subcore_axis_name="s")
    f = pl.kernel(
        _sc_chain_body,
        out_type=[jax.ShapeDtypeStruct((_NPAD, _NCLS), jnp.float32)] * 3
        + [jax.ShapeDtypeStruct((_HALF, _NCLS), jnp.float32)] * 2,
        mesh=mesh,
        compiler_params=pltpu.CompilerParams(use_tc_tiling_on_sc=False),
        scratch_types=[
            pltpu.VMEM((_CPW, _CHUNK), jnp.int32),
            pltpu.VMEM((_CPW, _CHUNK), jnp.int32),
            pltpu.VMEM((_CPW * _CHUNK + 16,), jnp.float32),
            pltpu.VMEM((_CHUNK, _NCLS), jnp.float32),
            pltpu.VMEM((_CHUNK, _NCLS), jnp.float32),
            pltpu.VMEM((_CHUNK, _NCLS), jnp.float32),
            pltpu.VMEM((_CHUNK, _NCLS), jnp.float32),
            pltpu.VMEM((_RPT, _NCLS), jnp.float32),
            pltpu.VMEM((_RPT, _NCLS), jnp.float32),
            pltpu.VMEM((_CMB, _NCLS), jnp.float32),
            pltpu.VMEM((_CMB, _NCLS), jnp.float32),
            pltpu.VMEM((_CMB, _NCLS), jnp.float32),
            pltpu.VMEM_SHARED((_NPAD, _NCLS), jnp.float32),
            pltpu.VMEM_SHARED((_NPAD, _NCLS), jnp.float32),
        ] + [pltpu.SemaphoreType.DMA] * 8
        + [pltpu.SemaphoreType.REGULAR],
    )
    return f(cols, rows, vals, m[0], m[1], m[2], m[3], m[4])[0]


def kernel(features, adj_values, upper_W, upper_b, bottom_W, bottom_b,
           fc_W, fc_b, adj_indices):
    f32 = jnp.float32
    features = features.astype(f32)
    cols = adj_indices[1].astype(jnp.int32).reshape(_WORKERS, _CPW, _CHUNK)
    rows = adj_indices[0].astype(jnp.int32).reshape(_WORKERS, _CPW, _CHUNK)
    vals = adj_values.astype(f32).reshape(_WORKERS, _CPW * _CHUNK)

    w_all = jnp.transpose(upper_W, (1, 0, 2)).reshape(_D, _A1)
    b_all = upper_b.reshape(1, _A1)
    bw = bottom_W.astype(f32)
    fc = fc_W.astype(f32)

    nblk = _N // _ROWBLK
    m = pl.pallas_call(
        _dense_body,
        grid=(nblk,),
        in_specs=[
            pl.BlockSpec((_ROWBLK, _D), lambda i: (i, 0)),
            pl.BlockSpec((_D, _A1), lambda i: (0, 0)),
            pl.BlockSpec((1, _A1), lambda i: (0, 0)),
            pl.BlockSpec((3, _A1, _L), lambda i: (0, 0, 0)),
            pl.BlockSpec((_A1, _NCLS), lambda i: (0, 0)),
        ],
        out_specs=[pl.BlockSpec((_ROWBLK, _NCLS), lambda i: (i, 0))] * _NM,
        out_shape=[jax.ShapeDtypeStruct((_NPAD, _NCLS), f32)] * _NM,
    )(features, w_all, b_all, bw, fc)

    p = _sc_chain(cols, rows, vals, m)

    out = pl.pallas_call(
        _finish_body,
        grid=(_N // _FINBLK,),
        in_specs=[
            pl.BlockSpec((_FINBLK, _NCLS), lambda i: (i, 0)),
            pl.BlockSpec((3, 1, _L), lambda i: (0, 0, 0)),
            pl.BlockSpec((_A1, _NCLS), lambda i: (0, 0)),
            pl.BlockSpec((1, _NCLS), lambda i: (0, 0)),
        ],
        out_specs=pl.BlockSpec((_FINBLK, _NCLS), lambda i: (i, 0)),
        out_shape=jax.ShapeDtypeStruct((_N, _NCLS), f32),
    )(p, bottom_b.astype(f32), fc, fc_b.astype(f32).reshape(1, _NCLS))
    return out


# speedup vs baseline: 51.8153x; 1.0004x over previous
import jax
import jax.numpy as jnp
from jax import lax
from jax.experimental import pallas as pl
from jax.experimental.pallas import tpu as pltpu
from jax.experimental.pallas import tpu_sc as plsc

_N = 10000
_E = 320000
_D = 128
_L = 200
_A1 = 600
_NCLS = 16
_NM = 5

_CHUNK = 125
_NCHUNKS = _E // _CHUNK
_TILES = 16
_WORKERS = 32
_CPW = _NCHUNKS // _WORKERS
_HALF = 5120
_CMB = 320
_NPAD = 10240
_RPT = _NPAD // _TILES
_ROWBLK = 2000
_FINBLK = 2000


def _make_q(bw_ref, fc_ref):
    g = [
        jnp.dot(bw_ref[i], fc_ref[_L * i:_L * (i + 1), :],
                preferred_element_type=jnp.float32)
        for i in range(3)
    ]
    zero = jnp.zeros((_L, _NCLS), jnp.float32)
    cols = []
    for k in range(_NM):
        parts = []
        for b in range(3):
            i = k - b
            parts.append(g[i][_L * b:_L * (b + 1), :] if 0 <= i <= 2 else zero)
        cols.append(jnp.concatenate(parts, axis=0))
    return jnp.concatenate(cols, axis=1)


def _dense_body(f_ref, w_ref, b_ref, bw_ref, fc_ref, *m_refs):
    q = _make_q(bw_ref, fc_ref)
    h = jnp.dot(f_ref[...].astype(jnp.bfloat16), w_ref[...].astype(jnp.bfloat16),
                preferred_element_type=jnp.float32)
    h = jnp.maximum(h + b_ref[...], 0.0)
    m = jnp.dot(h.astype(jnp.bfloat16), q.astype(jnp.bfloat16),
                preferred_element_type=jnp.float32)
    for k in range(_NM):
        m_refs[k][...] = m[:, _NCLS * k:_NCLS * (k + 1)]


def _finish_body(x_ref, bb_ref, fc_ref, fcb_ref, o_ref):
    bias = fcb_ref[...]
    for i in range(3):
        bias = bias + jnp.dot(bb_ref[i], fc_ref[_L * i:_L * (i + 1), :],
                              preferred_element_type=jnp.float32)
    z = x_ref[...] + bias
    z = z - jnp.max(z, axis=1, keepdims=True)
    o_ref[...] = z - jnp.log(jnp.sum(jnp.exp(z), axis=1, keepdims=True))


def _sc_chain_body(cols_ref, rows_ref, vals_ref, m0_ref, m1_ref, m2_ref,
                   m3_ref, m4_ref,
                   out_ref, pa_ref, pb_ref, ph0_ref, ph1_ref,
                   cols_v, rows_v, vals_v, rb0, rb1, rb2, rb3,
                   zbuf, xbuf, obuf, pbuf, mbuf, acc, xloc,
                   gs0, gs1, gs2, gs3, ss0, ss1, ss2, ss3, hs):
    cid = lax.axis_index("c")
    sid = lax.axis_index("s")
    w = cid * _TILES + sid
    zbase = sid * _RPT
    own = cid * _HALF + sid * _CMB
    exp = (1 - cid) * _HALF + sid * _CMB
    ps = sid * _CMB

    pltpu.sync_copy(cols_ref.at[w], cols_v)
    pltpu.sync_copy(rows_ref.at[w], rows_v)
    pltpu.sync_copy(vals_ref.at[w], vals_v.at[pl.ds(0, _CPW * _CHUNK)])

    def _zfill(i, _):
        zbuf[i, :] = jnp.zeros((16,), jnp.float32)
        return 0

    lax.fori_loop(0, _RPT, _zfill, 0)
    pltpu.sync_copy(m4_ref.at[pl.ds(sid * _RPT, _RPT)], xbuf)
    pltpu.sync_copy(xbuf, xloc.at[pl.ds(sid * _RPT, _RPT)])

    lane_ids = [jnp.full((16,), u, jnp.int32) for u in range(16)]
    rbs = [rb0, rb1, rb2, rb3]
    gss = [gs0, gs1, gs2, gs3]
    sss = [ss0, ss1, ss2, ss3]

    def handshake():
        plsc.subcore_barrier()

        @pl.when(sid == 0)
        def _hs():
            pl.semaphore_signal(hs, 1, core_index=1 - cid)
            pl.semaphore_wait(hs, 1)

        plsc.subcore_barrier()

    def scale(rb, c):
        for g in range(8):
            vv = vals_v[pl.ds(c * _CHUNK + g * 16, 16)]
            for u in range(16):
                e = g * 16 + u
                if e < _CHUNK:
                    rb[e, :] = rb[e, :] * jnp.take(vv, lane_ids[u])

    def hop(m_src, x_dst, tag, last):
      with jax.named_scope(tag):
        pltpu.sync_copy(zbuf, acc.at[pl.ds(zbase, _RPT)])
        plsc.subcore_barrier()
        pltpu.async_copy(xloc.at[cols_v.at[0]], rb0, gs0)
        pltpu.async_copy(xloc.at[cols_v.at[1]], rb1, gs1)

        def body(j, _):
            for b in range(4):
                c = 4 * j + b
                pltpu.make_async_copy(
                    xloc.at[cols_v.at[c]], rbs[b], gss[b]).wait()
                scale(rbs[b], c)
                pltpu.async_copy(
                    rbs[b], acc.at[rows_v.at[c]], sss[b], add=True)
                nb = (b + 2) % 4

                def _wait_prev(nb=nb, pc=c - 2):
                    pltpu.make_async_copy(
                        rbs[nb], acc.at[rows_v.at[pc]], sss[nb]).wait()

                def _issue(nb=nb, c2=c + 2):
                    pltpu.async_copy(
                        xloc.at[cols_v.at[c2]], rbs[nb], gss[nb])

                if b < 2:
                    pl.when(j > 0)(_wait_prev)
                    _issue()
                else:
                    _wait_prev()
                    pl.when(j < _CPW // 4 - 1)(_issue)
            return 0

        lax.fori_loop(0, _CPW // 4, body, 0)
        pltpu.make_async_copy(rb2, acc.at[rows_v.at[_CPW - 2]], ss2).wait()
        pltpu.make_async_copy(rb3, acc.at[rows_v.at[_CPW - 1]], ss3).wait()
        plsc.subcore_barrier()
      with jax.named_scope(tag + "_readout"):
        @pl.when(cid == 0)
        def _exp0():
            pltpu.sync_copy(acc.at[pl.ds(exp, _CMB)],
                            ph0_ref.at[pl.ds(ps, _CMB)])

        @pl.when(cid == 1)
        def _exp1():
            pltpu.sync_copy(acc.at[pl.ds(exp, _CMB)],
                            ph1_ref.at[pl.ds(ps, _CMB)])

        handshake()
        pltpu.sync_copy(acc.at[pl.ds(own, _CMB)], obuf)

        @pl.when(cid == 0)
        def _imp0():
            pltpu.sync_copy(ph1_ref.at[pl.ds(ps, _CMB)], pbuf)

        @pl.when(cid == 1)
        def _imp1():
            pltpu.sync_copy(ph0_ref.at[pl.ds(ps, _CMB)], pbuf)

        pltpu.sync_copy(m_src.at[pl.ds(own, _CMB)], mbuf)

        def radd(k, _):
            e0 = k * 5
            for u in range(5):
                e = e0 + u
                obuf[e, :] = (obuf[e, :] + pbuf[e, :]) + mbuf[e, :]
            return 0

        lax.fori_loop(0, _CMB // 5, radd, 0)
        pltpu.sync_copy(obuf, x_dst.at[pl.ds(own, _CMB)])
        if not last:
            pltpu.sync_copy(obuf, xloc.at[pl.ds(own, _CMB)])
            handshake()
            pltpu.sync_copy(x_dst.at[pl.ds(exp, _CMB)], pbuf)
            pltpu.sync_copy(pbuf, xloc.at[pl.ds(exp, _CMB)])

    hop(m3_ref, pa_ref, "hop1", False)
    hop(m2_ref, pb_ref, "hop2", False)
    hop(m1_ref, pa_ref, "hop3", False)
    hop(m0_ref, out_ref, "hop4", True)


def _sc_chain(cols, rows, vals, m):
    mesh = plsc.VectorSubcoreMesh(core_axis_name="c", subcore_axis_name="s")
    f = pl.kernel(
        _sc_chain_body,
        out_type=[jax.ShapeDtypeStruct((_NPAD, _NCLS), jnp.float32)] * 3
        + [jax.ShapeDtypeStruct((_HALF, _NCLS), jnp.float32)] * 2,
        mesh=mesh,
        compiler_params=pltpu.CompilerParams(use_tc_tiling_on_sc=False),
        scratch_types=[
            pltpu.VMEM((_CPW, _CHUNK), jnp.int32),
            pltpu.VMEM((_CPW, _CHUNK), jnp.int32),
            pltpu.VMEM((_CPW * _CHUNK + 16,), jnp.float32),
            pltpu.VMEM((_CHUNK, _NCLS), jnp.float32),
            pltpu.VMEM((_CHUNK, _NCLS), jnp.float32),
            pltpu.VMEM((_CHUNK, _NCLS), jnp.float32),
            pltpu.VMEM((_CHUNK, _NCLS), jnp.float32),
            pltpu.VMEM((_RPT, _NCLS), jnp.float32),
            pltpu.VMEM((_RPT, _NCLS), jnp.float32),
            pltpu.VMEM((_CMB, _NCLS), jnp.float32),
            pltpu.VMEM((_CMB, _NCLS), jnp.float32),
            pltpu.VMEM((_CMB, _NCLS), jnp.float32),
            pltpu.VMEM_SHARED((_NPAD, _NCLS), jnp.float32),
            pltpu.VMEM_SHARED((_NPAD, _NCLS), jnp.float32),
        ] + [pltpu.SemaphoreType.DMA] * 8
        + [pltpu.SemaphoreType.REGULAR],
    )
    return f(cols, rows, vals, m[0], m[1], m[2], m[3], m[4])[0]


def kernel(features, adj_values, upper_W, upper_b, bottom_W, bottom_b,
           fc_W, fc_b, adj_indices):
    f32 = jnp.float32
    features = features.astype(f32)
    cols = adj_indices[1].astype(jnp.int32).reshape(_WORKERS, _CPW, _CHUNK)
    rows = adj_indices[0].astype(jnp.int32).reshape(_WORKERS, _CPW, _CHUNK)
    vals = adj_values.astype(f32).reshape(_WORKERS, _CPW * _CHUNK)

    w_all = jnp.transpose(upper_W, (1, 0, 2)).reshape(_D, _A1)
    b_all = upper_b.reshape(1, _A1)
    bw = bottom_W.astype(f32)
    fc = fc_W.astype(f32)

    nblk = _N // _ROWBLK
    m = pl.pallas_call(
        _dense_body,
        grid=(nblk,),
        in_specs=[
            pl.BlockSpec((_ROWBLK, _D), lambda i: (i, 0)),
            pl.BlockSpec((_D, _A1), lambda i: (0, 0)),
            pl.BlockSpec((1, _A1), lambda i: (0, 0)),
            pl.BlockSpec((3, _A1, _L), lambda i: (0, 0, 0)),
            pl.BlockSpec((_A1, _NCLS), lambda i: (0, 0)),
        ],
        out_specs=[pl.BlockSpec((_ROWBLK, _NCLS), lambda i: (i, 0))] * _NM,
        out_shape=[jax.ShapeDtypeStruct((_NPAD, _NCLS), f32)] * _NM,
    )(features, w_all, b_all, bw, fc)

    p = _sc_chain(cols, rows, vals, m)

    out = pl.pallas_call(
        _finish_body,
        grid=(_N // _FINBLK,),
        in_specs=[
            pl.BlockSpec((_FINBLK, _NCLS), lambda i: (i, 0)),
            pl.BlockSpec((3, 1, _L), lambda i: (0, 0, 0)),
            pl.BlockSpec((_A1, _NCLS), lambda i: (0, 0)),
            pl.BlockSpec((1, _NCLS), lambda i: (0, 0)),
        ],
        out_specs=pl.BlockSpec((_FINBLK, _NCLS), lambda i: (i, 0)),
        out_shape=jax.ShapeDtypeStruct((_N, _NCLS), f32),
    )(p, bottom_b.astype(f32), fc, fc_b.astype(f32).reshape(1, _NCLS))
    return out
